# baseline (speedup 1.0000x reference)
import functools

import jax
import jax.numpy as jnp
from jax import lax
from jax.experimental import pallas as pl
from jax.experimental.pallas import tpu as pltpu
from jax.experimental.pallas import tpu_sc as plsc

K = 20
EMBED = 128
EPS = 1e-5

NC, NS, L = 2, 16, 16
NW = NC * NS
B, N = 16, 2048
CB = 32
NCB = EMBED // CB
PPW = (B * N) // NW
NCHUNK = 256
NGRP = NCHUNK // L
NCH = PPW // NCHUNK
NSTAT = 5
CAP = 128
NSLOT = CAP // L
NEGINF = float("-inf")
MAXI = 2**30


def _prep_body(xyz_ref, w_ref, ut_ref, v_ref, xx_ref):
    x = xyz_ref[0]
    w = w_ref[...]
    w1 = w[:, 0:3]
    w2 = w[:, 3:6]
    dn = (((1,), (1,)), ((), ()))
    ut = lax.dot_general(w1 - w2, x, dn, preferred_element_type=jnp.float32)
    ut_ref[0] = ut
    v = lax.dot_general(x, w2, dn, preferred_element_type=jnp.float32)
    for cb in range(NCB):
        v_ref[0, cb] = v[:, cb * CB:(cb + 1) * CB]
    xx_ref[0] = jnp.sum(x * x, axis=1).reshape(1, N)


def _prep(xyz, W):
    f = pl.pallas_call(
        _prep_body,
        grid=(B,),
        in_specs=[
            pl.BlockSpec((1, N, 3), lambda b: (b, 0, 0)),
            pl.BlockSpec((EMBED, 6), lambda b: (0, 0)),
        ],
        out_specs=[
            pl.BlockSpec((1, EMBED, N), lambda b: (b, 0, 0)),
            pl.BlockSpec((1, NCB, N, CB), lambda b: (b, 0, 0, 0)),
            pl.BlockSpec((1, 1, N), lambda b: (b, 0, 0)),
        ],
        out_shape=[
            jax.ShapeDtypeStruct((B, EMBED, N), jnp.float32),
            jax.ShapeDtypeStruct((B, NCB, N, CB), jnp.float32),
            jax.ShapeDtypeStruct((B, 1, N), jnp.float32),
        ],
    )
    return f(xyz, W)


TR = 256


def _pd_body(xt_ref, xf_ref, xxc_ref, xxr_ref, o_ref):
    xt = xt_ref[0]
    xf = xf_ref[0]
    dn = (((1,), (1,)), ((), ()))
    mm = lax.dot_general(xt, xf, dn, preferred_element_type=jnp.float32)
    inner = -2.0 * mm
    o_ref[0] = (-xxc_ref[0]) - inner - xxr_ref[0]


def _pairwise(xyz, xx):
    xxc = xx.reshape(B, N, 1)
    f = pl.pallas_call(
        _pd_body,
        grid=(B, N // TR),
        in_specs=[
            pl.BlockSpec((1, TR, 3), lambda b, i: (b, i, 0)),
            pl.BlockSpec((1, N, 3), lambda b, i: (b, 0, 0)),
            pl.BlockSpec((1, TR, 1), lambda b, i: (b, i, 0)),
            pl.BlockSpec((1, 1, N), lambda b, i: (b, 0, 0)),
        ],
        out_specs=pl.BlockSpec((1, TR, N), lambda b, i: (b, i, 0)),
        out_shape=jax.ShapeDtypeStruct((B, N, N), jnp.float32),
    )
    return f(xyz, xyz, xxc, xx)


def _shuf(v, idx):
    return jnp.take_along_axis(v, idx, axis=0, mode="promise_in_bounds")


def _lane_max(v):
    iota = lax.iota(jnp.int32, L)
    for dist in (8, 4, 2, 1):
        v = jnp.maximum(v, _shuf(v, jnp.bitwise_xor(iota, dist)))
    return v


def _lane_min_i(v):
    iota = lax.iota(jnp.int32, L)
    for dist in (8, 4, 2, 1):
        v = jnp.minimum(v, _shuf(v, jnp.bitwise_xor(iota, dist)))
    return v


def _bitonic_clean_asc(v):
    iota = lax.iota(jnp.int32, L)
    for dist in (8, 4, 2, 1):
        w = _shuf(v, jnp.bitwise_xor(iota, dist))
        up = (jnp.bitwise_and(iota, dist) == 0)
        v = jnp.where(up, jnp.minimum(v, w), jnp.maximum(v, w))
    return v


def _merge2_asc(a, b):
    bl = b[::-1]
    lo = _bitonic_clean_asc(jnp.minimum(a, bl))
    hi = _bitonic_clean_asc(jnp.maximum(a, bl))
    return lo, hi


def _process_row(rowref, cidbuf, r, dbuf, jbuf, wbuf, oidx):
    iota = lax.iota(jnp.int32, L)
    neg = jnp.full((L,), NEGINF, jnp.float32)
    iota16 = iota * L

    cms = []
    for q in range(8):
        base = q * (L * L) + iota16
        cm = neg
        for e in range(L):
            cm = jnp.maximum(cm, plsc.load_gather(rowref, [base + e]))
        cms.append(cm)

    M0 = jnp.maximum(jnp.maximum(cms[0], cms[1]), jnp.maximum(cms[2], cms[3]))
    M1 = jnp.maximum(jnp.maximum(cms[4], cms[5]), jnp.maximum(cms[6], cms[7]))
    sA = jnp.sort(M0)
    sB = jnp.sort(M1)
    lo = jnp.minimum(sA, sB[::-1])
    lo = _bitonic_clean_asc(lo)
    t0 = _shuf(lo, jnp.full((L,), 11, jnp.int32))

    cw = jnp.zeros((L,), jnp.int32)
    for q in range(8):
        m = cms[q] >= t0
        pos = jnp.minimum(cw + plsc.cumsum(m.astype(jnp.int32)) - 1, CAP - 1)
        plsc.store_scatter(cidbuf, [pos], q * L + iota, mask=m)
        cw = cw + plsc.all_reduce_population_count(m)
    nacc_v = cw
    ng = (jnp.max(cw) + (L - 1)) // L

    for s in range(NSLOT):
        dbuf[pl.ds(s * L, L)] = neg

    def s2(q2, wp):
        cid = cidbuf[pl.ds(q2 * L, L)]
        valid = (q2 * L + iota) < nacc_v
        addr = cid * L
        run = wp
        for h in range(2):
            gs, ms, bases = [], [], []
            for e in range(8):
                g = plsc.load_gather(rowref, [addr + (h * 8 + e)])
                ms.append((g >= t0) & valid)
                gs.append(g)
            for e in range(8):
                bases.append(run)
                run = run + plsc.all_reduce_population_count(ms[e])
            for e in range(8):
                pos = jnp.minimum(
                    bases[e] + plsc.cumsum(ms[e].astype(jnp.int32)) - 1,
                    CAP - 1)
                plsc.store_scatter(dbuf, [pos], gs[e], mask=ms[e])
                plsc.store_scatter(jbuf, [pos], addr + (h * 8 + e),
                                   mask=ms[e])
        return run

    wp = lax.fori_loop(0, ng, s2, jnp.zeros((L,), jnp.int32))
    wps = jnp.max(wp)
    ns = (wps + (L - 1)) // L

    def _extract_sort():
        s0 = jnp.sort(dbuf[pl.ds(0, L)])
        s1_ = jnp.sort(dbuf[pl.ds(L, L)])
        s2_ = jnp.sort(dbuf[pl.ds(2 * L, L)])
        s3 = jnp.sort(dbuf[pl.ds(3 * L, L)])
        l0, h0 = _merge2_asc(s0, s1_)
        l1, h1 = _merge2_asc(s2_, s3)
        M0u = jnp.maximum(l0, h1[::-1])
        M1u = jnp.maximum(h0, l1[::-1])
        au = _bitonic_clean_asc(jnp.minimum(M0u, M1u))
        t20v = _shuf(au, jnp.full((L,), 12, jnp.int32))
        cnt = jnp.zeros((L,), jnp.int32)
        for s in range(4):
            cnt = cnt + plsc.all_reduce_population_count(
                dbuf[pl.ds(s * L, L)] > t20v)
        return jnp.max(cnt), jnp.max(t20v)

    def _extract_iter():
        for s in range(NSLOT):
            wbuf[pl.ds(s * L, L)] = dbuf[pl.ds(s * L, L)]

        def xcond(st):
            removed, _, _ = st
            return removed < K

        def xbody(st):
            removed, _, _ = st

            def slotmax(s, acc):
                return jnp.maximum(acc, wbuf[pl.ds(s * L, L)])

            mv = lax.fori_loop(0, ns, slotmax, neg)
            sv = _lane_max(mv)

            def rem(s, cnt):
                d = wbuf[pl.ds(s * L, L)]
                m = d == sv
                wbuf[pl.ds(s * L, L)] = jnp.where(m, neg, d)
                return cnt + plsc.all_reduce_population_count(m)

            cntv = lax.fori_loop(0, ns, rem, jnp.zeros((L,), jnp.int32))
            return removed + jnp.max(cntv), removed, jnp.max(sv)

        removed, a0, t20s0 = lax.while_loop(
            xcond, xbody, (jnp.int32(0), jnp.int32(0), jnp.float32(NEGINF)))
        del removed
        return a0, t20s0

    a, t20s = lax.cond(wps <= 4 * L, _extract_sort, _extract_iter)
    need = K - a
    t20 = jnp.full((L,), t20s, jnp.float32)

    def tcond(st):
        taken, _ = st
        return taken < need

    def tbody(st):
        taken, jprev = st

        def slotmin(s, acc):
            d = dbuf[pl.ds(s * L, L)]
            j = jbuf[pl.ds(s * L, L)]
            m = (d == t20) & (j > jprev)
            return jnp.minimum(acc, jnp.where(m, j, MAXI))

        jm = lax.fori_loop(0, ns, slotmin, jnp.full((L,), MAXI, jnp.int32))
        return taken + 1, jnp.min(jm)

    _, jcut = lax.while_loop(tcond, tbody, (jnp.int32(0), jnp.int32(-1)))
    jcut_v = jnp.full((L,), jcut, jnp.int32)

    base20 = r * K

    def outs(s, run):
        d = dbuf[pl.ds(s * L, L)]
        j = jbuf[pl.ds(s * L, L)]
        sel = (d > t20) | ((d == t20) & (j <= jcut_v))
        pos = base20 + run + plsc.cumsum(sel.astype(jnp.int32)) - 1
        plsc.store_scatter(oidx, [pos], j, mask=sel)
        return run + plsc.all_reduce_population_count(sel)

    lax.fori_loop(0, ns, outs, jnp.zeros((L,), jnp.int32))


def _sc_topk_body(pd_hbm, idx_hbm, row0, row1, dbuf, jbuf, wbuf, cidbuf,
                  oidx, sem0, sem1):
    wid = lax.axis_index("s") * NC + lax.axis_index("c")
    b = wid // 2
    half = wid % 2
    base = half * PPW
    npair = PPW // 2

    for s in range(NSLOT):
        cidbuf[pl.ds(s * L, L)] = jnp.zeros((L,), jnp.int32)

    pltpu.async_copy(pd_hbm.at[b, base + 0], row0, sem0)
    pltpu.async_copy(pd_hbm.at[b, base + 1], row1, sem1)

    def pair(p, _):
        r0 = 2 * p
        r1 = 2 * p + 1
        pltpu.make_async_copy(pd_hbm.at[b, base + r0], row0, sem0).wait()
        _process_row(row0, cidbuf, r0, dbuf, jbuf, wbuf, oidx)

        @pl.when(p < npair - 1)
        def _():
            pltpu.async_copy(pd_hbm.at[b, base + r0 + 2], row0, sem0)

        pltpu.make_async_copy(pd_hbm.at[b, base + r1], row1, sem1).wait()
        _process_row(row1, cidbuf, r1, dbuf, jbuf, wbuf, oidx)

        @pl.when(p < npair - 1)
        def _():
            pltpu.async_copy(pd_hbm.at[b, base + r1 + 2], row1, sem1)

        return 0

    lax.fori_loop(0, npair, pair, 0)
    pltpu.sync_copy(oidx, idx_hbm.at[b, pl.ds(base * K, PPW * K)])


def _sc_topk(pd):
    mesh = plsc.VectorSubcoreMesh(core_axis_name="c", subcore_axis_name="s")
    f = pl.kernel(
        _sc_topk_body,
        out_type=jax.ShapeDtypeStruct((B, N * K), jnp.int32),
        mesh=mesh,
        compiler_params=pltpu.CompilerParams(needs_layout_passes=False),
        scratch_types=[
            pltpu.VMEM((N,), jnp.float32),
            pltpu.VMEM((N,), jnp.float32),
            pltpu.VMEM((CAP,), jnp.float32),
            pltpu.VMEM((CAP,), jnp.int32),
            pltpu.VMEM((CAP,), jnp.float32),
            pltpu.VMEM((CAP,), jnp.int32),
            pltpu.VMEM((PPW * K,), jnp.int32),
            pltpu.SemaphoreType.DMA,
            pltpu.SemaphoreType.DMA,
        ],
    )
    return f(pd)


def _sc_gather_reduce(v_hbm, ut_hbm, idx_hbm,
                      mx_hbm, mn_hbm, part_hbm,
                      vblk, idxv, ucbuf, omx, omn, acc):
    wid = lax.axis_index("s") * NC + lax.axis_index("c")
    b = wid // 2
    half = wid % 2
    base = half * PPW

    def _z(i, _):
        acc[pl.ds(i * L, L)] = jnp.zeros((L,), jnp.float32)
        return 0
    lax.fori_loop(0, EMBED * NSTAT, _z, 0)

    pltpu.sync_copy(idx_hbm.at[b, pl.ds(base * K, PPW * K)], idxv)

    iota = lax.iota(jnp.int32, L)

    for cb in range(NCB):
        pltpu.sync_copy(v_hbm.at[b, cb], vblk)
        for ch in range(NCH):
            noff = ch * NCHUNK
            pltpu.sync_copy(
                ut_hbm.at[b, cb, :, pl.ds(base + noff, NCHUNK)], ucbuf)

            def grp_body(gi, _, cb=cb, noff=noff):
                nvec = (noff + gi * L + iota) * K
                jvc = [plsc.load_gather(idxv, [nvec + k]) * CB
                       for k in range(K)]

                @plsc.parallel_loop(0, CB, unroll=2)
                def c_body(c):
                    splat_c = jnp.full((L,), c, jnp.int32)
                    acc4 = [[jnp.zeros((L,), jnp.float32),
                             jnp.zeros((L,), jnp.float32),
                             jnp.full((L,), -jnp.inf, jnp.float32),
                             jnp.full((L,), jnp.inf, jnp.float32)]
                            for _ in range(4)]
                    for k in range(K):
                        g = plsc.load_gather(vblk, [jvc[k] + splat_c])
                        p = acc4[k % 4]
                        p[0] = p[0] + g
                        p[1] = p[1] + g * g
                        p[2] = jnp.maximum(p[2], g)
                        p[3] = jnp.minimum(p[3], g)
                    s1 = (acc4[0][0] + acc4[1][0]) + (acc4[2][0] + acc4[3][0])
                    s2 = (acc4[0][1] + acc4[1][1]) + (acc4[2][1] + acc4[3][1])
                    mx = jnp.maximum(jnp.maximum(acc4[0][2], acc4[1][2]),
                                     jnp.maximum(acc4[2][2], acc4[3][2]))
                    mn = jnp.minimum(jnp.minimum(acc4[0][3], acc4[1][3]),
                                     jnp.minimum(acc4[2][3], acc4[3][3]))
                    u = ucbuf[c, pl.ds(gi * L, L)]
                    arow = (cb * CB + c) * (NSTAT * L)
                    acc[pl.ds(arow, L)] = acc[pl.ds(arow, L)] + s1
                    acc[pl.ds(arow + L, L)] = acc[pl.ds(arow + L, L)] + s2
                    acc[pl.ds(arow + 2 * L, L)] = acc[pl.ds(arow + 2 * L, L)] + u * s1
                    acc[pl.ds(arow + 3 * L, L)] = acc[pl.ds(arow + 3 * L, L)] + u
                    acc[pl.ds(arow + 4 * L, L)] = acc[pl.ds(arow + 4 * L, L)] + u * u
                    omx[c, pl.ds(gi * L, L)] = mx
                    omn[c, pl.ds(gi * L, L)] = mn

                return 0

            lax.fori_loop(0, NGRP, grp_body, 0)
            pltpu.sync_copy(
                omx, mx_hbm.at[b, cb, :, pl.ds(base + noff, NCHUNK)])
            pltpu.sync_copy(
                omn, mn_hbm.at[b, cb, :, pl.ds(base + noff, NCHUNK)])

    pltpu.sync_copy(acc, part_hbm.at[wid])


def _gather_reduce(v_blk, ut_blk, idx):
    mesh = plsc.VectorSubcoreMesh(core_axis_name="c", subcore_axis_name="s")
    f = pl.kernel(
        _sc_gather_reduce,
        out_type=[
            jax.ShapeDtypeStruct((B, NCB, CB, N), jnp.float32),
            jax.ShapeDtypeStruct((B, NCB, CB, N), jnp.float32),
            jax.ShapeDtypeStruct((NW, EMBED * NSTAT * L), jnp.float32),
        ],
        mesh=mesh,
        compiler_params=pltpu.CompilerParams(needs_layout_passes=False),
        scratch_types=[
            pltpu.VMEM((N * CB,), jnp.float32),
            pltpu.VMEM((PPW * K,), jnp.int32),
            pltpu.VMEM((CB, NCHUNK), jnp.float32),
            pltpu.VMEM((CB, NCHUNK), jnp.float32),
            pltpu.VMEM((CB, NCHUNK), jnp.float32),
            pltpu.VMEM((EMBED * NSTAT * L,), jnp.float32),
        ],
    )
    return f(v_blk, ut_blk, idx)


def _combine_body(u_ref, vmax_ref, vmin_ref, g_ref, c_ref, o_ref):
    u = u_ref[...]
    g = g_ref[...]
    c = c_ref[...]
    hmax = (u + vmax_ref[...]) * g + c
    hmin = (u + vmin_ref[...]) * g + c

    def hswish(y):
        return y * jnp.clip(y + 3.0, 0.0, 6.0) * (1.0 / 6.0)

    o_ref[...] = jnp.maximum(hswish(hmax), hswish(hmin))


def _combine(UT, VmaxT, VminT, g, c):
    Bb, E, Nn = UT.shape
    TN = 512
    grid = (Bb, Nn // TN)
    in_spec = pl.BlockSpec((1, E, TN), lambda b, i: (b, 0, i))
    vec_spec = pl.BlockSpec((E, 1), lambda b, i: (0, 0))
    out_spec = pl.BlockSpec((1, E, TN), lambda b, i: (b, 0, i))
    f = pl.pallas_call(
        lambda u, vx, vn, gg, cc, o: _combine_body(
            u.at[0], vx.at[0], vn.at[0], gg, cc, o.at[0]),
        grid=grid,
        in_specs=[in_spec, in_spec, in_spec, vec_spec, vec_spec],
        out_specs=out_spec,
        out_shape=jax.ShapeDtypeStruct((Bb, E, Nn), jnp.float32),
    )
    return f(UT, VmaxT, VminT, g.reshape(E, 1), c.reshape(E, 1))


def kernel(xyz, W, gamma, beta):
    ut, vblk4, xx = _prep(xyz, W)
    pd = _pairwise(xyz, xx)
    idx = _sc_topk(pd)

    v_blk = vblk4.reshape(B, NCB, N * CB)
    ut_blk = ut.reshape(B, NCB, CB, N)
    mx, mn, part = _gather_reduce(v_blk, ut_blk, idx)

    sums = jnp.sum(part.reshape(NW, EMBED, NSTAT, L), axis=(0, 3))
    s_S1, s_S2, s_US1, s_U, s_U2 = (sums[:, i] for i in range(NSTAT))
    cnt = B * N * K
    mean = (K * s_U + s_S1) / cnt
    var = (K * s_U2 + 2.0 * s_US1 + s_S2) / cnt - mean * mean
    g = gamma / jnp.sqrt(var + EPS)
    c = beta - g * mean

    x = _combine(ut, mx.reshape(B, EMBED, N), mn.reshape(B, EMBED, N), g, c)
    return (xyz, x)


# speedup vs baseline: 1.0164x; 1.0164x over previous
import functools

import jax
import jax.numpy as jnp
from jax import lax
from jax.experimental import pallas as pl
from jax.experimental.pallas import tpu as pltpu
from jax.experimental.pallas import tpu_sc as plsc

K = 20
EMBED = 128
EPS = 1e-5

NC, NS, L = 2, 16, 16
NW = NC * NS
B, N = 16, 2048
CB = 32
NCB = EMBED // CB
PPW = (B * N) // NW
NCHUNK = 256
NGRP = NCHUNK // L
NCH = PPW // NCHUNK
NSTAT = 5
CAP = 128
NSLOT = CAP // L
NEGINF = float("-inf")
MAXI = 2**30


def _prep_body(xyz_ref, w_ref, ut_ref, v_ref, xx_ref):
    x = xyz_ref[0]
    w = w_ref[...]
    w1 = w[:, 0:3]
    w2 = w[:, 3:6]
    dn = (((1,), (1,)), ((), ()))
    ut = lax.dot_general(w1 - w2, x, dn, preferred_element_type=jnp.float32)
    ut_ref[0] = ut
    v = lax.dot_general(x, w2, dn, preferred_element_type=jnp.float32)
    for cb in range(NCB):
        v_ref[0, cb] = v[:, cb * CB:(cb + 1) * CB]
    xx_ref[0] = jnp.sum(x * x, axis=1).reshape(1, N)


def _prep(xyz, W):
    f = pl.pallas_call(
        _prep_body,
        grid=(B,),
        in_specs=[
            pl.BlockSpec((1, N, 3), lambda b: (b, 0, 0)),
            pl.BlockSpec((EMBED, 6), lambda b: (0, 0)),
        ],
        out_specs=[
            pl.BlockSpec((1, EMBED, N), lambda b: (b, 0, 0)),
            pl.BlockSpec((1, NCB, N, CB), lambda b: (b, 0, 0, 0)),
            pl.BlockSpec((1, 1, N), lambda b: (b, 0, 0)),
        ],
        out_shape=[
            jax.ShapeDtypeStruct((B, EMBED, N), jnp.float32),
            jax.ShapeDtypeStruct((B, NCB, N, CB), jnp.float32),
            jax.ShapeDtypeStruct((B, 1, N), jnp.float32),
        ],
    )
    return f(xyz, W)


TR = 256


def _pd_body(xt_ref, xf_ref, xxc_ref, xxr_ref, o_ref):
    xt = xt_ref[0]
    xf = xf_ref[0]
    dn = (((1,), (1,)), ((), ()))
    mm = lax.dot_general(xt, xf, dn, preferred_element_type=jnp.float32)
    inner = -2.0 * mm
    o_ref[0] = (-xxc_ref[0]) - inner - xxr_ref[0]


def _pairwise(xyz, xx):
    xxc = xx.reshape(B, N, 1)
    f = pl.pallas_call(
        _pd_body,
        grid=(B, N // TR),
        in_specs=[
            pl.BlockSpec((1, TR, 3), lambda b, i: (b, i, 0)),
            pl.BlockSpec((1, N, 3), lambda b, i: (b, 0, 0)),
            pl.BlockSpec((1, TR, 1), lambda b, i: (b, i, 0)),
            pl.BlockSpec((1, 1, N), lambda b, i: (b, 0, 0)),
        ],
        out_specs=pl.BlockSpec((1, TR, N), lambda b, i: (b, i, 0)),
        out_shape=jax.ShapeDtypeStruct((B, N, N), jnp.float32),
    )
    return f(xyz, xyz, xxc, xx)


def _shuf(v, idx):
    return jnp.take_along_axis(v, idx, axis=0, mode="promise_in_bounds")


def _lane_max(v):
    iota = lax.iota(jnp.int32, L)
    for dist in (8, 4, 2, 1):
        v = jnp.maximum(v, _shuf(v, jnp.bitwise_xor(iota, dist)))
    return v


def _lane_min_i(v):
    iota = lax.iota(jnp.int32, L)
    for dist in (8, 4, 2, 1):
        v = jnp.minimum(v, _shuf(v, jnp.bitwise_xor(iota, dist)))
    return v


def _bitonic_clean_asc(v):
    iota = lax.iota(jnp.int32, L)
    for dist in (8, 4, 2, 1):
        w = _shuf(v, jnp.bitwise_xor(iota, dist))
        up = (jnp.bitwise_and(iota, dist) == 0)
        v = jnp.where(up, jnp.minimum(v, w), jnp.maximum(v, w))
    return v


def _merge2_asc(a, b):
    bl = b[::-1]
    lo = _bitonic_clean_asc(jnp.minimum(a, bl))
    hi = _bitonic_clean_asc(jnp.maximum(a, bl))
    return lo, hi


def _process_row(rowref, cidbuf, r, dbuf, jbuf, wbuf, oidx):
    iota = lax.iota(jnp.int32, L)
    neg = jnp.full((L,), NEGINF, jnp.float32)
    iota16 = iota * L

    cms = []
    for q in range(8):
        base = q * (L * L) + iota16
        cm = neg
        for e in range(L):
            cm = jnp.maximum(cm, plsc.load_gather(rowref, [base + e]))
        cms.append(cm)

    M0 = jnp.maximum(jnp.maximum(cms[0], cms[1]), jnp.maximum(cms[2], cms[3]))
    M1 = jnp.maximum(jnp.maximum(cms[4], cms[5]), jnp.maximum(cms[6], cms[7]))
    sA = jnp.sort(M0)
    sB = jnp.sort(M1)
    lo = jnp.minimum(sA, sB[::-1])
    lo = _bitonic_clean_asc(lo)
    t0 = _shuf(lo, jnp.full((L,), 11, jnp.int32))

    cw = jnp.zeros((L,), jnp.int32)
    for q in range(8):
        m = cms[q] >= t0
        pos = jnp.minimum(cw + plsc.cumsum(m.astype(jnp.int32)) - 1, CAP - 1)
        plsc.store_scatter(cidbuf, [pos], q * L + iota, mask=m)
        cw = cw + plsc.all_reduce_population_count(m)
    nacc_v = cw
    ng = (jnp.max(cw) + (L - 1)) // L

    for s in range(NSLOT):
        dbuf[pl.ds(s * L, L)] = neg

    def s2(q2, wp):
        cid = cidbuf[pl.ds(q2 * L, L)]
        valid = (q2 * L + iota) < nacc_v
        addr = cid * L
        run = wp
        for h in range(2):
            gs, ms, bases = [], [], []
            for e in range(8):
                g = plsc.load_gather(rowref, [addr + (h * 8 + e)])
                ms.append((g >= t0) & valid)
                gs.append(g)
            for e in range(8):
                bases.append(run)
                run = run + plsc.all_reduce_population_count(ms[e])
            for e in range(8):
                pos = jnp.minimum(
                    bases[e] + plsc.cumsum(ms[e].astype(jnp.int32)) - 1,
                    CAP - 1)
                plsc.store_scatter(dbuf, [pos], gs[e], mask=ms[e])
                plsc.store_scatter(jbuf, [pos], addr + (h * 8 + e),
                                   mask=ms[e])
        return run

    wp = lax.fori_loop(0, ng, s2, jnp.zeros((L,), jnp.int32))
    wps = jnp.max(wp)
    ns = (wps + (L - 1)) // L

    def _extract_sort():
        s0 = jnp.sort(dbuf[pl.ds(0, L)])
        s1_ = jnp.sort(dbuf[pl.ds(L, L)])
        s2_ = jnp.sort(dbuf[pl.ds(2 * L, L)])
        s3 = jnp.sort(dbuf[pl.ds(3 * L, L)])
        l0, h0 = _merge2_asc(s0, s1_)
        l1, h1 = _merge2_asc(s2_, s3)
        M0u = jnp.maximum(l0, h1[::-1])
        M1u = jnp.maximum(h0, l1[::-1])
        au = _bitonic_clean_asc(jnp.minimum(M0u, M1u))
        t20v = _shuf(au, jnp.full((L,), 12, jnp.int32))
        cnt = jnp.zeros((L,), jnp.int32)
        for s in range(4):
            cnt = cnt + plsc.all_reduce_population_count(
                dbuf[pl.ds(s * L, L)] > t20v)
        return jnp.max(cnt), jnp.max(t20v)

    def _extract_iter():
        for s in range(NSLOT):
            wbuf[pl.ds(s * L, L)] = dbuf[pl.ds(s * L, L)]

        def xcond(st):
            removed, _, _ = st
            return removed < K

        def xbody(st):
            removed, _, _ = st

            def slotmax(s, acc):
                return jnp.maximum(acc, wbuf[pl.ds(s * L, L)])

            mv = lax.fori_loop(0, ns, slotmax, neg)
            sv = _lane_max(mv)

            def rem(s, cnt):
                d = wbuf[pl.ds(s * L, L)]
                m = d == sv
                wbuf[pl.ds(s * L, L)] = jnp.where(m, neg, d)
                return cnt + plsc.all_reduce_population_count(m)

            cntv = lax.fori_loop(0, ns, rem, jnp.zeros((L,), jnp.int32))
            return removed + jnp.max(cntv), removed, jnp.max(sv)

        removed, a0, t20s0 = lax.while_loop(
            xcond, xbody, (jnp.int32(0), jnp.int32(0), jnp.float32(NEGINF)))
        del removed
        return a0, t20s0

    a, t20s = lax.cond(wps <= 4 * L, _extract_sort, _extract_iter)
    need = K - a
    t20 = jnp.full((L,), t20s, jnp.float32)

    def tcond(st):
        taken, _ = st
        return taken < need

    def tbody(st):
        taken, jprev = st

        def slotmin(s, acc):
            d = dbuf[pl.ds(s * L, L)]
            j = jbuf[pl.ds(s * L, L)]
            m = (d == t20) & (j > jprev)
            return jnp.minimum(acc, jnp.where(m, j, MAXI))

        jm = lax.fori_loop(0, ns, slotmin, jnp.full((L,), MAXI, jnp.int32))
        return taken + 1, jnp.min(jm)

    _, jcut = lax.while_loop(tcond, tbody, (jnp.int32(0), jnp.int32(-1)))
    jcut_v = jnp.full((L,), jcut, jnp.int32)

    base20 = r * K

    def outs(s, run):
        d = dbuf[pl.ds(s * L, L)]
        j = jbuf[pl.ds(s * L, L)]
        sel = (d > t20) | ((d == t20) & (j <= jcut_v))
        pos = base20 + run + plsc.cumsum(sel.astype(jnp.int32)) - 1
        plsc.store_scatter(oidx, [pos], j, mask=sel)
        return run + plsc.all_reduce_population_count(sel)

    lax.fori_loop(0, ns, outs, jnp.zeros((L,), jnp.int32))


def _sc_topk_body(pd_hbm, idx_hbm, row0, row1, dbuf, jbuf, wbuf, cidbuf,
                  oidx, sem0, sem1):
    wid = lax.axis_index("s") * NC + lax.axis_index("c")
    b = wid // 2
    half = wid % 2
    base = half * PPW
    npair = PPW // 2

    for s in range(NSLOT):
        cidbuf[pl.ds(s * L, L)] = jnp.zeros((L,), jnp.int32)

    pltpu.async_copy(pd_hbm.at[b, base + 0], row0, sem0)
    pltpu.async_copy(pd_hbm.at[b, base + 1], row1, sem1)

    def pair(p, _):
        r0 = 2 * p
        r1 = 2 * p + 1
        pltpu.make_async_copy(pd_hbm.at[b, base + r0], row0, sem0).wait()
        _process_row(row0, cidbuf, r0, dbuf, jbuf, wbuf, oidx)

        @pl.when(p < npair - 1)
        def _():
            pltpu.async_copy(pd_hbm.at[b, base + r0 + 2], row0, sem0)

        pltpu.make_async_copy(pd_hbm.at[b, base + r1], row1, sem1).wait()
        _process_row(row1, cidbuf, r1, dbuf, jbuf, wbuf, oidx)

        @pl.when(p < npair - 1)
        def _():
            pltpu.async_copy(pd_hbm.at[b, base + r1 + 2], row1, sem1)

        return 0

    lax.fori_loop(0, npair, pair, 0)
    pltpu.sync_copy(oidx, idx_hbm.at[b, pl.ds(base * K, PPW * K)])


def _sc_topk(pd):
    mesh = plsc.VectorSubcoreMesh(core_axis_name="c", subcore_axis_name="s")
    f = pl.kernel(
        _sc_topk_body,
        out_type=jax.ShapeDtypeStruct((B, N * K), jnp.int32),
        mesh=mesh,
        compiler_params=pltpu.CompilerParams(needs_layout_passes=False),
        scratch_types=[
            pltpu.VMEM((N,), jnp.float32),
            pltpu.VMEM((N,), jnp.float32),
            pltpu.VMEM((CAP,), jnp.float32),
            pltpu.VMEM((CAP,), jnp.int32),
            pltpu.VMEM((CAP,), jnp.float32),
            pltpu.VMEM((CAP,), jnp.int32),
            pltpu.VMEM((PPW * K,), jnp.int32),
            pltpu.SemaphoreType.DMA,
            pltpu.SemaphoreType.DMA,
        ],
    )
    return f(pd)


def _sc_gather_reduce(v_hbm, ut_hbm, idx_hbm,
                      mx_hbm, mn_hbm, part_hbm,
                      vblk, idxv, ucbuf, omx, omn, acc):
    wid = lax.axis_index("s") * NC + lax.axis_index("c")
    b = wid // 2
    half = wid % 2
    base = half * PPW

    def _z(i, _):
        acc[pl.ds(i * L, L)] = jnp.zeros((L,), jnp.float32)
        return 0
    lax.fori_loop(0, EMBED * NSTAT, _z, 0)

    pltpu.sync_copy(idx_hbm.at[b, pl.ds(base * K, PPW * K)], idxv)

    iota = lax.iota(jnp.int32, L)

    for cb in range(NCB):
        pltpu.sync_copy(v_hbm.at[b, cb], vblk)
        for ch in range(NCH):
            noff = ch * NCHUNK
            pltpu.sync_copy(
                ut_hbm.at[b, cb, :, pl.ds(base + noff, NCHUNK)], ucbuf)

            def grp_body(gi, _, cb=cb, noff=noff):
                nvec = (noff + gi * L + iota) * K
                jvc = [plsc.load_gather(idxv, [nvec + k]) * CB
                       for k in range(K)]

                @plsc.parallel_loop(0, CB, unroll=2)
                def c_body(c):
                    splat_c = jnp.full((L,), c, jnp.int32)
                    acc4 = [[jnp.zeros((L,), jnp.float32),
                             jnp.zeros((L,), jnp.float32),
                             jnp.full((L,), -jnp.inf, jnp.float32),
                             jnp.full((L,), jnp.inf, jnp.float32)]
                            for _ in range(2)]
                    for k in range(K):
                        g = plsc.load_gather(vblk, [jvc[k] + splat_c])
                        p = acc4[k % 2]
                        p[0] = p[0] + g
                        p[1] = p[1] + g * g
                        p[2] = jnp.maximum(p[2], g)
                        p[3] = jnp.minimum(p[3], g)
                    s1 = acc4[0][0] + acc4[1][0]
                    s2 = acc4[0][1] + acc4[1][1]
                    mx = jnp.maximum(acc4[0][2], acc4[1][2])
                    mn = jnp.minimum(acc4[0][3], acc4[1][3])
                    u = ucbuf[c, pl.ds(gi * L, L)]
                    arow = (cb * CB + c) * (NSTAT * L)
                    acc[pl.ds(arow, L)] = acc[pl.ds(arow, L)] + s1
                    acc[pl.ds(arow + L, L)] = acc[pl.ds(arow + L, L)] + s2
                    acc[pl.ds(arow + 2 * L, L)] = acc[pl.ds(arow + 2 * L, L)] + u * s1
                    acc[pl.ds(arow + 3 * L, L)] = acc[pl.ds(arow + 3 * L, L)] + u
                    acc[pl.ds(arow + 4 * L, L)] = acc[pl.ds(arow + 4 * L, L)] + u * u
                    omx[c, pl.ds(gi * L, L)] = mx
                    omn[c, pl.ds(gi * L, L)] = mn

                return 0

            lax.fori_loop(0, NGRP, grp_body, 0)
            pltpu.sync_copy(
                omx, mx_hbm.at[b, cb, :, pl.ds(base + noff, NCHUNK)])
            pltpu.sync_copy(
                omn, mn_hbm.at[b, cb, :, pl.ds(base + noff, NCHUNK)])

    pltpu.sync_copy(acc, part_hbm.at[wid])


def _gather_reduce(v_blk, ut_blk, idx):
    mesh = plsc.VectorSubcoreMesh(core_axis_name="c", subcore_axis_name="s")
    f = pl.kernel(
        _sc_gather_reduce,
        out_type=[
            jax.ShapeDtypeStruct((B, NCB, CB, N), jnp.float32),
            jax.ShapeDtypeStruct((B, NCB, CB, N), jnp.float32),
            jax.ShapeDtypeStruct((NW, EMBED * NSTAT * L), jnp.float32),
        ],
        mesh=mesh,
        compiler_params=pltpu.CompilerParams(needs_layout_passes=False),
        scratch_types=[
            pltpu.VMEM((N * CB,), jnp.float32),
            pltpu.VMEM((PPW * K,), jnp.int32),
            pltpu.VMEM((CB, NCHUNK), jnp.float32),
            pltpu.VMEM((CB, NCHUNK), jnp.float32),
            pltpu.VMEM((CB, NCHUNK), jnp.float32),
            pltpu.VMEM((EMBED * NSTAT * L,), jnp.float32),
        ],
    )
    return f(v_blk, ut_blk, idx)


def _combine_body(u_ref, vmax_ref, vmin_ref, g_ref, c_ref, o_ref):
    u = u_ref[...]
    g = g_ref[...]
    c = c_ref[...]
    hmax = (u + vmax_ref[...]) * g + c
    hmin = (u + vmin_ref[...]) * g + c

    def hswish(y):
        return y * jnp.clip(y + 3.0, 0.0, 6.0) * (1.0 / 6.0)

    o_ref[...] = jnp.maximum(hswish(hmax), hswish(hmin))


def _combine(UT, VmaxT, VminT, g, c):
    Bb, E, Nn = UT.shape
    TN = 512
    grid = (Bb, Nn // TN)
    in_spec = pl.BlockSpec((1, E, TN), lambda b, i: (b, 0, i))
    vec_spec = pl.BlockSpec((E, 1), lambda b, i: (0, 0))
    out_spec = pl.BlockSpec((1, E, TN), lambda b, i: (b, 0, i))
    f = pl.pallas_call(
        lambda u, vx, vn, gg, cc, o: _combine_body(
            u.at[0], vx.at[0], vn.at[0], gg, cc, o.at[0]),
        grid=grid,
        in_specs=[in_spec, in_spec, in_spec, vec_spec, vec_spec],
        out_specs=out_spec,
        out_shape=jax.ShapeDtypeStruct((Bb, E, Nn), jnp.float32),
    )
    return f(UT, VmaxT, VminT, g.reshape(E, 1), c.reshape(E, 1))


def kernel(xyz, W, gamma, beta):
    ut, vblk4, xx = _prep(xyz, W)
    pd = _pairwise(xyz, xx)
    idx = _sc_topk(pd)

    v_blk = vblk4.reshape(B, NCB, N * CB)
    ut_blk = ut.reshape(B, NCB, CB, N)
    mx, mn, part = _gather_reduce(v_blk, ut_blk, idx)

    sums = jnp.sum(part.reshape(NW, EMBED, NSTAT, L), axis=(0, 3))
    s_S1, s_S2, s_US1, s_U, s_U2 = (sums[:, i] for i in range(NSTAT))
    cnt = B * N * K
    mean = (K * s_U + s_S1) / cnt
    var = (K * s_U2 + 2.0 * s_US1 + s_S2) / cnt - mean * mean
    g = gamma / jnp.sqrt(var + EPS)
    c = beta - g * mean

    x = _combine(ut, mx.reshape(B, EMBED, N), mn.reshape(B, EMBED, N), g, c)
    return (xyz, x)


# speedup vs baseline: 1.0798x; 1.0624x over previous
import functools

import jax
import jax.numpy as jnp
from jax import lax
from jax.experimental import pallas as pl
from jax.experimental.pallas import tpu as pltpu
from jax.experimental.pallas import tpu_sc as plsc

K = 20
EMBED = 128
EPS = 1e-5

NC, NS, L = 2, 16, 16
NW = NC * NS
B, N = 16, 2048
CB = 32
NCB = EMBED // CB
PPW = (B * N) // NW
NCHUNK = 256
NGRP = NCHUNK // L
NCH = PPW // NCHUNK
NSTAT = 5
CAP = 128
NSLOT = CAP // L
NEGINF = float("-inf")
MAXI = 2**30


def _prep_body(xyz_ref, w_ref, ut_ref, v_ref, xx_ref, su_ref, su2_ref):
    x = xyz_ref[0]
    w = w_ref[...]
    w1 = w[:, 0:3]
    w2 = w[:, 3:6]
    dn = (((1,), (1,)), ((), ()))
    ut = lax.dot_general(w1 - w2, x, dn, preferred_element_type=jnp.float32)
    ut_ref[0] = ut
    v = lax.dot_general(x, w2, dn, preferred_element_type=jnp.float32)
    for cb in range(NCB):
        v_ref[0, cb] = v[:, cb * CB:(cb + 1) * CB]
    xx_ref[0] = jnp.sum(x * x, axis=1).reshape(1, N)
    su_ref[0] = jnp.sum(ut, axis=1).reshape(1, EMBED)
    su2_ref[0] = jnp.sum(ut * ut, axis=1).reshape(1, EMBED)


def _prep(xyz, W):
    f = pl.pallas_call(
        _prep_body,
        grid=(B,),
        in_specs=[
            pl.BlockSpec((1, N, 3), lambda b: (b, 0, 0)),
            pl.BlockSpec((EMBED, 6), lambda b: (0, 0)),
        ],
        out_specs=[
            pl.BlockSpec((1, EMBED, N), lambda b: (b, 0, 0)),
            pl.BlockSpec((1, NCB, N, CB), lambda b: (b, 0, 0, 0)),
            pl.BlockSpec((1, 1, N), lambda b: (b, 0, 0)),
            pl.BlockSpec((1, 1, EMBED), lambda b: (b, 0, 0)),
            pl.BlockSpec((1, 1, EMBED), lambda b: (b, 0, 0)),
        ],
        out_shape=[
            jax.ShapeDtypeStruct((B, EMBED, N), jnp.float32),
            jax.ShapeDtypeStruct((B, NCB, N, CB), jnp.float32),
            jax.ShapeDtypeStruct((B, 1, N), jnp.float32),
            jax.ShapeDtypeStruct((B, 1, EMBED), jnp.float32),
            jax.ShapeDtypeStruct((B, 1, EMBED), jnp.float32),
        ],
    )
    return f(xyz, W)


def _cstats_body(cnt_ref, v_ref, cv_ref, cv2_ref):
    cnt = cnt_ref[0]
    dn = (((1,), (0,)), ((), ()))
    for cb in range(NCB):
        vb = v_ref[0, cb]
        cv_ref[0, 0, cb * CB:(cb + 1) * CB] = lax.dot_general(
            cnt, vb, dn, preferred_element_type=jnp.float32)[0]
        cv2_ref[0, 0, cb * CB:(cb + 1) * CB] = lax.dot_general(
            cnt, vb * vb, dn, preferred_element_type=jnp.float32)[0]


def _cstats(cnt, vblk4):
    f = pl.pallas_call(
        _cstats_body,
        grid=(B,),
        in_specs=[
            pl.BlockSpec((1, 1, N), lambda b: (b, 0, 0)),
            pl.BlockSpec((1, NCB, N, CB), lambda b: (b, 0, 0, 0)),
        ],
        out_specs=[
            pl.BlockSpec((1, 1, EMBED), lambda b: (b, 0, 0)),
            pl.BlockSpec((1, 1, EMBED), lambda b: (b, 0, 0)),
        ],
        out_shape=[
            jax.ShapeDtypeStruct((B, 1, EMBED), jnp.float32),
            jax.ShapeDtypeStruct((B, 1, EMBED), jnp.float32),
        ],
    )
    return f(cnt, vblk4)


TR = 256


def _pd_body(xt_ref, xf_ref, xxc_ref, xxr_ref, o_ref):
    xt = xt_ref[0]
    xf = xf_ref[0]
    dn = (((1,), (1,)), ((), ()))
    mm = lax.dot_general(xt, xf, dn, preferred_element_type=jnp.float32)
    inner = -2.0 * mm
    o_ref[0] = (-xxc_ref[0]) - inner - xxr_ref[0]


def _pairwise(xyz, xx):
    xxc = xx.reshape(B, N, 1)
    f = pl.pallas_call(
        _pd_body,
        grid=(B, N // TR),
        in_specs=[
            pl.BlockSpec((1, TR, 3), lambda b, i: (b, i, 0)),
            pl.BlockSpec((1, N, 3), lambda b, i: (b, 0, 0)),
            pl.BlockSpec((1, TR, 1), lambda b, i: (b, i, 0)),
            pl.BlockSpec((1, 1, N), lambda b, i: (b, 0, 0)),
        ],
        out_specs=pl.BlockSpec((1, TR, N), lambda b, i: (b, i, 0)),
        out_shape=jax.ShapeDtypeStruct((B, N, N), jnp.float32),
    )
    return f(xyz, xyz, xxc, xx)


def _shuf(v, idx):
    return jnp.take_along_axis(v, idx, axis=0, mode="promise_in_bounds")


def _lane_max(v):
    iota = lax.iota(jnp.int32, L)
    for dist in (8, 4, 2, 1):
        v = jnp.maximum(v, _shuf(v, jnp.bitwise_xor(iota, dist)))
    return v


def _lane_min_i(v):
    iota = lax.iota(jnp.int32, L)
    for dist in (8, 4, 2, 1):
        v = jnp.minimum(v, _shuf(v, jnp.bitwise_xor(iota, dist)))
    return v


def _bitonic_clean_asc(v):
    iota = lax.iota(jnp.int32, L)
    for dist in (8, 4, 2, 1):
        w = _shuf(v, jnp.bitwise_xor(iota, dist))
        up = (jnp.bitwise_and(iota, dist) == 0)
        v = jnp.where(up, jnp.minimum(v, w), jnp.maximum(v, w))
    return v


def _merge2_asc(a, b):
    bl = b[::-1]
    lo = _bitonic_clean_asc(jnp.minimum(a, bl))
    hi = _bitonic_clean_asc(jnp.maximum(a, bl))
    return lo, hi


def _process_row(rowref, cidbuf, r, dbuf, jbuf, wbuf, oidx):
    iota = lax.iota(jnp.int32, L)
    neg = jnp.full((L,), NEGINF, jnp.float32)
    iota16 = iota * L

    cms = []
    for q in range(8):
        base = q * (L * L) + iota16
        cm = neg
        for e in range(L):
            cm = jnp.maximum(cm, plsc.load_gather(rowref, [base + e]))
        cms.append(cm)

    M0 = jnp.maximum(jnp.maximum(cms[0], cms[1]), jnp.maximum(cms[2], cms[3]))
    M1 = jnp.maximum(jnp.maximum(cms[4], cms[5]), jnp.maximum(cms[6], cms[7]))
    sA = jnp.sort(M0)
    sB = jnp.sort(M1)
    lo = jnp.minimum(sA, sB[::-1])
    lo = _bitonic_clean_asc(lo)
    t0 = _shuf(lo, jnp.full((L,), 11, jnp.int32))

    cw = jnp.zeros((L,), jnp.int32)
    for q in range(8):
        m = cms[q] >= t0
        pos = jnp.minimum(cw + plsc.cumsum(m.astype(jnp.int32)) - 1, CAP - 1)
        plsc.store_scatter(cidbuf, [pos], q * L + iota, mask=m)
        cw = cw + plsc.all_reduce_population_count(m)
    nacc_v = cw
    ng = (jnp.max(cw) + (L - 1)) // L

    for s in range(NSLOT):
        dbuf[pl.ds(s * L, L)] = neg

    def s2(q2, wp):
        cid = cidbuf[pl.ds(q2 * L, L)]
        valid = (q2 * L + iota) < nacc_v
        addr = cid * L
        run = wp
        for h in range(2):
            gs, ms, bases = [], [], []
            for e in range(8):
                g = plsc.load_gather(rowref, [addr + (h * 8 + e)])
                ms.append((g >= t0) & valid)
                gs.append(g)
            for e in range(8):
                bases.append(run)
                run = run + plsc.all_reduce_population_count(ms[e])
            for e in range(8):
                pos = jnp.minimum(
                    bases[e] + plsc.cumsum(ms[e].astype(jnp.int32)) - 1,
                    CAP - 1)
                plsc.store_scatter(dbuf, [pos], gs[e], mask=ms[e])
                plsc.store_scatter(jbuf, [pos], addr + (h * 8 + e),
                                   mask=ms[e])
        return run

    wp = lax.fori_loop(0, ng, s2, jnp.zeros((L,), jnp.int32))
    wps = jnp.max(wp)
    ns = (wps + (L - 1)) // L

    def _extract_sort():
        s0 = jnp.sort(dbuf[pl.ds(0, L)])
        s1_ = jnp.sort(dbuf[pl.ds(L, L)])
        s2_ = jnp.sort(dbuf[pl.ds(2 * L, L)])
        s3 = jnp.sort(dbuf[pl.ds(3 * L, L)])
        l0, h0 = _merge2_asc(s0, s1_)
        l1, h1 = _merge2_asc(s2_, s3)
        M0u = jnp.maximum(l0, h1[::-1])
        M1u = jnp.maximum(h0, l1[::-1])
        au = _bitonic_clean_asc(jnp.minimum(M0u, M1u))
        t20v = _shuf(au, jnp.full((L,), 12, jnp.int32))
        cnt = jnp.zeros((L,), jnp.int32)
        for s in range(4):
            cnt = cnt + plsc.all_reduce_population_count(
                dbuf[pl.ds(s * L, L)] > t20v)
        return jnp.max(cnt), jnp.max(t20v)

    def _extract_iter():
        for s in range(NSLOT):
            wbuf[pl.ds(s * L, L)] = dbuf[pl.ds(s * L, L)]

        def xcond(st):
            removed, _, _ = st
            return removed < K

        def xbody(st):
            removed, _, _ = st

            def slotmax(s, acc):
                return jnp.maximum(acc, wbuf[pl.ds(s * L, L)])

            mv = lax.fori_loop(0, ns, slotmax, neg)
            sv = _lane_max(mv)

            def rem(s, cnt):
                d = wbuf[pl.ds(s * L, L)]
                m = d == sv
                wbuf[pl.ds(s * L, L)] = jnp.where(m, neg, d)
                return cnt + plsc.all_reduce_population_count(m)

            cntv = lax.fori_loop(0, ns, rem, jnp.zeros((L,), jnp.int32))
            return removed + jnp.max(cntv), removed, jnp.max(sv)

        removed, a0, t20s0 = lax.while_loop(
            xcond, xbody, (jnp.int32(0), jnp.int32(0), jnp.float32(NEGINF)))
        del removed
        return a0, t20s0

    a, t20s = lax.cond(wps <= 4 * L, _extract_sort, _extract_iter)
    need = K - a
    t20 = jnp.full((L,), t20s, jnp.float32)

    def tcond(st):
        taken, _ = st
        return taken < need

    def tbody(st):
        taken, jprev = st

        def slotmin(s, acc):
            d = dbuf[pl.ds(s * L, L)]
            j = jbuf[pl.ds(s * L, L)]
            m = (d == t20) & (j > jprev)
            return jnp.minimum(acc, jnp.where(m, j, MAXI))

        jm = lax.fori_loop(0, ns, slotmin, jnp.full((L,), MAXI, jnp.int32))
        return taken + 1, jnp.min(jm)

    _, jcut = lax.while_loop(tcond, tbody, (jnp.int32(0), jnp.int32(-1)))
    jcut_v = jnp.full((L,), jcut, jnp.int32)

    base20 = r * K

    def outs(s, run):
        d = dbuf[pl.ds(s * L, L)]
        j = jbuf[pl.ds(s * L, L)]
        sel = (d > t20) | ((d == t20) & (j <= jcut_v))
        pos = base20 + run + plsc.cumsum(sel.astype(jnp.int32)) - 1
        plsc.store_scatter(oidx, [pos], j, mask=sel)
        return run + plsc.all_reduce_population_count(sel)

    lax.fori_loop(0, ns, outs, jnp.zeros((L,), jnp.int32))


def _sc_topk_body(pd_hbm, idx_hbm, row0, row1, dbuf, jbuf, wbuf, cidbuf,
                  oidx, sem0, sem1):
    wid = lax.axis_index("s") * NC + lax.axis_index("c")
    b = wid // 2
    half = wid % 2
    base = half * PPW
    npair = PPW // 2

    for s in range(NSLOT):
        cidbuf[pl.ds(s * L, L)] = jnp.zeros((L,), jnp.int32)

    pltpu.async_copy(pd_hbm.at[b, base + 0], row0, sem0)
    pltpu.async_copy(pd_hbm.at[b, base + 1], row1, sem1)

    def pair(p, _):
        r0 = 2 * p
        r1 = 2 * p + 1
        pltpu.make_async_copy(pd_hbm.at[b, base + r0], row0, sem0).wait()
        _process_row(row0, cidbuf, r0, dbuf, jbuf, wbuf, oidx)

        @pl.when(p < npair - 1)
        def _():
            pltpu.async_copy(pd_hbm.at[b, base + r0 + 2], row0, sem0)

        pltpu.make_async_copy(pd_hbm.at[b, base + r1], row1, sem1).wait()
        _process_row(row1, cidbuf, r1, dbuf, jbuf, wbuf, oidx)

        @pl.when(p < npair - 1)
        def _():
            pltpu.async_copy(pd_hbm.at[b, base + r1 + 2], row1, sem1)

        return 0

    lax.fori_loop(0, npair, pair, 0)
    pltpu.sync_copy(oidx, idx_hbm.at[b, pl.ds(base * K, PPW * K)])


def _sc_topk(pd):
    mesh = plsc.VectorSubcoreMesh(core_axis_name="c", subcore_axis_name="s")
    f = pl.kernel(
        _sc_topk_body,
        out_type=jax.ShapeDtypeStruct((B, N * K), jnp.int32),
        mesh=mesh,
        compiler_params=pltpu.CompilerParams(needs_layout_passes=False),
        scratch_types=[
            pltpu.VMEM((N,), jnp.float32),
            pltpu.VMEM((N,), jnp.float32),
            pltpu.VMEM((CAP,), jnp.float32),
            pltpu.VMEM((CAP,), jnp.int32),
            pltpu.VMEM((CAP,), jnp.float32),
            pltpu.VMEM((CAP,), jnp.int32),
            pltpu.VMEM((PPW * K,), jnp.int32),
            pltpu.SemaphoreType.DMA,
            pltpu.SemaphoreType.DMA,
        ],
    )
    return f(pd)


def _sc_gather_reduce(v_hbm, ut_hbm, idx_hbm,
                      mx_hbm, mn_hbm, part_hbm, cnt_hbm,
                      vblk, idxv, ucbuf, omx, omn, acc, cntb):
    wid = lax.axis_index("s") * NC + lax.axis_index("c")
    b = wid // 2
    half = wid % 2
    base = half * PPW

    def _z(i, _):
        acc[pl.ds(i * L, L)] = jnp.zeros((L,), jnp.float32)
        return 0
    lax.fori_loop(0, EMBED, _z, 0)

    def _zc(i, _):
        cntb[pl.ds(i * L, L)] = jnp.zeros((L,), jnp.float32)
        return 0
    lax.fori_loop(0, N // L, _zc, 0)

    pltpu.sync_copy(idx_hbm.at[b, pl.ds(base * K, PPW * K)], idxv)

    iota = lax.iota(jnp.int32, L)
    ones = jnp.ones((L,), jnp.float32)

    for cb in range(NCB):
        pltpu.sync_copy(v_hbm.at[b, cb], vblk)
        for ch in range(NCH):
            noff = ch * NCHUNK
            pltpu.sync_copy(
                ut_hbm.at[b, cb, :, pl.ds(base + noff, NCHUNK)], ucbuf)

            def grp_body(gi, _, cb=cb, noff=noff):
                nvec = (noff + gi * L + iota) * K
                jv = [plsc.load_gather(idxv, [nvec + k]) for k in range(K)]
                if cb == 0:
                    for k in range(K):
                        plsc.addupdate_scatter(cntb, [jv[k]], ones)
                jvc = [j * CB for j in jv]

                @plsc.parallel_loop(0, CB, unroll=2)
                def c_body(c):
                    splat_c = jnp.full((L,), c, jnp.int32)
                    u = ucbuf[c, pl.ds(gi * L, L)]
                    cr = jnp.zeros((L,), jnp.float32)
                    mx = jnp.full((L,), -jnp.inf, jnp.float32)
                    mn = jnp.full((L,), jnp.inf, jnp.float32)
                    for k in range(K):
                        g = plsc.load_gather(vblk, [jvc[k] + splat_c])
                        cr = cr + u * g
                        mx = jnp.maximum(mx, g)
                        mn = jnp.minimum(mn, g)
                    arow = (cb * CB + c) * L
                    acc[pl.ds(arow, L)] = acc[pl.ds(arow, L)] + cr
                    omx[c, pl.ds(gi * L, L)] = mx
                    omn[c, pl.ds(gi * L, L)] = mn

                return 0

            lax.fori_loop(0, NGRP, grp_body, 0)
            pltpu.sync_copy(
                omx, mx_hbm.at[b, cb, :, pl.ds(base + noff, NCHUNK)])
            pltpu.sync_copy(
                omn, mn_hbm.at[b, cb, :, pl.ds(base + noff, NCHUNK)])

    pltpu.sync_copy(acc, part_hbm.at[wid])
    pltpu.sync_copy(cntb, cnt_hbm.at[wid])


def _gather_reduce(v_blk, ut_blk, idx):
    mesh = plsc.VectorSubcoreMesh(core_axis_name="c", subcore_axis_name="s")
    f = pl.kernel(
        _sc_gather_reduce,
        out_type=[
            jax.ShapeDtypeStruct((B, NCB, CB, N), jnp.float32),
            jax.ShapeDtypeStruct((B, NCB, CB, N), jnp.float32),
            jax.ShapeDtypeStruct((NW, EMBED * L), jnp.float32),
            jax.ShapeDtypeStruct((NW, N), jnp.float32),
        ],
        mesh=mesh,
        compiler_params=pltpu.CompilerParams(needs_layout_passes=False),
        scratch_types=[
            pltpu.VMEM((N * CB,), jnp.float32),
            pltpu.VMEM((PPW * K,), jnp.int32),
            pltpu.VMEM((CB, NCHUNK), jnp.float32),
            pltpu.VMEM((CB, NCHUNK), jnp.float32),
            pltpu.VMEM((CB, NCHUNK), jnp.float32),
            pltpu.VMEM((EMBED * L,), jnp.float32),
            pltpu.VMEM((N,), jnp.float32),
        ],
    )
    return f(v_blk, ut_blk, idx)


def _combine_body(u_ref, vmax_ref, vmin_ref, g_ref, c_ref, o_ref):
    u = u_ref[...]
    g = g_ref[...]
    c = c_ref[...]
    hmax = (u + vmax_ref[...]) * g + c
    hmin = (u + vmin_ref[...]) * g + c

    def hswish(y):
        return y * jnp.clip(y + 3.0, 0.0, 6.0) * (1.0 / 6.0)

    o_ref[...] = jnp.maximum(hswish(hmax), hswish(hmin))


def _combine(UT, VmaxT, VminT, g, c):
    Bb, E, Nn = UT.shape
    TN = 512
    grid = (Bb, Nn // TN)
    in_spec = pl.BlockSpec((1, E, TN), lambda b, i: (b, 0, i))
    vec_spec = pl.BlockSpec((E, 1), lambda b, i: (0, 0))
    out_spec = pl.BlockSpec((1, E, TN), lambda b, i: (b, 0, i))
    f = pl.pallas_call(
        lambda u, vx, vn, gg, cc, o: _combine_body(
            u.at[0], vx.at[0], vn.at[0], gg, cc, o.at[0]),
        grid=grid,
        in_specs=[in_spec, in_spec, in_spec, vec_spec, vec_spec],
        out_specs=out_spec,
        out_shape=jax.ShapeDtypeStruct((Bb, E, Nn), jnp.float32),
    )
    return f(UT, VmaxT, VminT, g.reshape(E, 1), c.reshape(E, 1))


def kernel(xyz, W, gamma, beta):
    ut, vblk4, xx, su, su2 = _prep(xyz, W)
    pd = _pairwise(xyz, xx)
    idx = _sc_topk(pd)

    v_blk = vblk4.reshape(B, NCB, N * CB)
    ut_blk = ut.reshape(B, NCB, CB, N)
    mx, mn, part, cntw = _gather_reduce(v_blk, ut_blk, idx)

    cntb = (cntw[0::2, :] + cntw[1::2, :]).reshape(B, 1, N)
    cv, cv2 = _cstats(cntb, vblk4)
    s_S1 = jnp.sum(cv[:, 0], axis=0)
    s_S2 = jnp.sum(cv2[:, 0], axis=0)
    s_US1 = jnp.sum(part.reshape(NW, EMBED, L), axis=(0, 2))
    s_U = jnp.sum(su[:, 0], axis=0)
    s_U2 = jnp.sum(su2[:, 0], axis=0)
    cnt = B * N * K
    mean = (K * s_U + s_S1) / cnt
    var = (K * s_U2 + 2.0 * s_US1 + s_S2) / cnt - mean * mean
    g = gamma / jnp.sqrt(var + EPS)
    c = beta - g * mean

    x = _combine(ut, mx.reshape(B, EMBED, N), mn.reshape(B, EMBED, N), g, c)
    return (xyz, x)


# speedup vs baseline: 1.7859x; 1.6539x over previous
import functools

import jax
import jax.numpy as jnp
from jax import lax
from jax.experimental import pallas as pl
from jax.experimental.pallas import tpu as pltpu
from jax.experimental.pallas import tpu_sc as plsc

K = 20
EMBED = 128
EPS = 1e-5

NC, NS, L = 2, 16, 16
NW = NC * NS
B, N = 16, 2048
CB = 32
NCB = EMBED // CB
PPW = (B * N) // NW
NCHUNK = 256
NGRP = NCHUNK // L
NCH = PPW // NCHUNK
NSTAT = 5
CAP = 128
NSLOT = CAP // L
NEGINF = float("-inf")
MAXI = 2**30


def _prep_body(xyz_ref, w_ref, ut_ref, v_ref, xx_ref, su_ref, su2_ref):
    x = xyz_ref[0]
    w = w_ref[...]
    w1 = w[:, 0:3]
    w2 = w[:, 3:6]
    dn = (((1,), (1,)), ((), ()))
    ut = lax.dot_general(w1 - w2, x, dn, preferred_element_type=jnp.float32)
    ut_ref[0] = ut
    v_ref[0] = lax.dot_general(w2, x, dn, preferred_element_type=jnp.float32)
    xx_ref[0] = jnp.sum(x * x, axis=1).reshape(1, N)
    su_ref[0] = jnp.sum(ut, axis=1).reshape(1, EMBED)
    su2_ref[0] = jnp.sum(ut * ut, axis=1).reshape(1, EMBED)


def _prep(xyz, W):
    f = pl.pallas_call(
        _prep_body,
        grid=(B,),
        in_specs=[
            pl.BlockSpec((1, N, 3), lambda b: (b, 0, 0)),
            pl.BlockSpec((EMBED, 6), lambda b: (0, 0)),
        ],
        out_specs=[
            pl.BlockSpec((1, EMBED, N), lambda b: (b, 0, 0)),
            pl.BlockSpec((1, EMBED, N), lambda b: (b, 0, 0)),
            pl.BlockSpec((1, 1, N), lambda b: (b, 0, 0)),
            pl.BlockSpec((1, 1, EMBED), lambda b: (b, 0, 0)),
            pl.BlockSpec((1, 1, EMBED), lambda b: (b, 0, 0)),
        ],
        out_shape=[
            jax.ShapeDtypeStruct((B, EMBED, N), jnp.float32),
            jax.ShapeDtypeStruct((B, EMBED, N), jnp.float32),
            jax.ShapeDtypeStruct((B, 1, N), jnp.float32),
            jax.ShapeDtypeStruct((B, 1, EMBED), jnp.float32),
            jax.ShapeDtypeStruct((B, 1, EMBED), jnp.float32),
        ],
    )
    return f(xyz, W)


def _cstats_body(cnt_ref, v_ref, cv_ref, cv2_ref):
    cnt = cnt_ref[0]
    vt = v_ref[0]
    dn = (((1,), (1,)), ((), ()))
    cv_ref[0] = lax.dot_general(vt, cnt, dn,
                                preferred_element_type=jnp.float32)
    cv2_ref[0] = lax.dot_general(vt * vt, cnt, dn,
                                 preferred_element_type=jnp.float32)


def _cstats(cnt, vt):
    f = pl.pallas_call(
        _cstats_body,
        grid=(B,),
        in_specs=[
            pl.BlockSpec((1, 1, N), lambda b: (b, 0, 0)),
            pl.BlockSpec((1, EMBED, N), lambda b: (b, 0, 0)),
        ],
        out_specs=[
            pl.BlockSpec((1, EMBED, 1), lambda b: (b, 0, 0)),
            pl.BlockSpec((1, EMBED, 1), lambda b: (b, 0, 0)),
        ],
        out_shape=[
            jax.ShapeDtypeStruct((B, EMBED, 1), jnp.float32),
            jax.ShapeDtypeStruct((B, EMBED, 1), jnp.float32),
        ],
    )
    return f(cnt, vt)


TR = 256


def _pd_body(xt_ref, xf_ref, xxc_ref, xxr_ref, o_ref):
    xt = xt_ref[0]
    xf = xf_ref[0]
    dn = (((1,), (1,)), ((), ()))
    mm = lax.dot_general(xt, xf, dn, preferred_element_type=jnp.float32)
    inner = -2.0 * mm
    o_ref[0] = (-xxc_ref[0]) - inner - xxr_ref[0]


def _pairwise(xyz, xx):
    xxc = xx.reshape(B, N, 1)
    f = pl.pallas_call(
        _pd_body,
        grid=(B, N // TR),
        in_specs=[
            pl.BlockSpec((1, TR, 3), lambda b, i: (b, i, 0)),
            pl.BlockSpec((1, N, 3), lambda b, i: (b, 0, 0)),
            pl.BlockSpec((1, TR, 1), lambda b, i: (b, i, 0)),
            pl.BlockSpec((1, 1, N), lambda b, i: (b, 0, 0)),
        ],
        out_specs=pl.BlockSpec((1, TR, N), lambda b, i: (b, i, 0)),
        out_shape=jax.ShapeDtypeStruct((B, N, N), jnp.float32),
    )
    return f(xyz, xyz, xxc, xx)


def _shuf(v, idx):
    return jnp.take_along_axis(v, idx, axis=0, mode="promise_in_bounds")


def _lane_max(v):
    iota = lax.iota(jnp.int32, L)
    for dist in (8, 4, 2, 1):
        v = jnp.maximum(v, _shuf(v, jnp.bitwise_xor(iota, dist)))
    return v


def _lane_min_i(v):
    iota = lax.iota(jnp.int32, L)
    for dist in (8, 4, 2, 1):
        v = jnp.minimum(v, _shuf(v, jnp.bitwise_xor(iota, dist)))
    return v


def _bitonic_clean_asc(v):
    iota = lax.iota(jnp.int32, L)
    for dist in (8, 4, 2, 1):
        w = _shuf(v, jnp.bitwise_xor(iota, dist))
        up = (jnp.bitwise_and(iota, dist) == 0)
        v = jnp.where(up, jnp.minimum(v, w), jnp.maximum(v, w))
    return v


def _merge2_asc(a, b):
    bl = b[::-1]
    lo = _bitonic_clean_asc(jnp.minimum(a, bl))
    hi = _bitonic_clean_asc(jnp.maximum(a, bl))
    return lo, hi


P17 = L + 1


def _process_row(rowref, rowpad, cidbuf, r, dbuf, jbuf, wbuf, oidx):
    iota = lax.iota(jnp.int32, L)
    neg = jnp.full((L,), NEGINF, jnp.float32)
    iota17 = iota * P17

    def rp(q, _):
        v = rowref[pl.ds(q * L, L)]
        plsc.store_scatter(rowpad, [q * P17 + iota], v)
        return 0

    lax.fori_loop(0, N // L, rp, 0)

    cms = []
    for q in range(8):
        base = q * (L * P17) + iota17
        cm = neg
        for e in range(L):
            cm = jnp.maximum(cm, plsc.load_gather(rowpad, [base + e]))
        cms.append(cm)

    M0 = jnp.maximum(jnp.maximum(cms[0], cms[1]), jnp.maximum(cms[2], cms[3]))
    M1 = jnp.maximum(jnp.maximum(cms[4], cms[5]), jnp.maximum(cms[6], cms[7]))
    sA = jnp.sort(M0)
    sB = jnp.sort(M1)
    lo = jnp.minimum(sA, sB[::-1])
    lo = _bitonic_clean_asc(lo)
    t0 = _shuf(lo, jnp.full((L,), 11, jnp.int32))

    cw = jnp.zeros((L,), jnp.int32)
    for q in range(8):
        m = cms[q] >= t0
        pos = jnp.minimum(cw + plsc.cumsum(m.astype(jnp.int32)) - 1, CAP - 1)
        plsc.store_scatter(cidbuf, [pos], q * L + iota, mask=m)
        cw = cw + plsc.all_reduce_population_count(m)
    nacc_v = cw
    ng = (jnp.max(cw) + (L - 1)) // L

    for s in range(NSLOT):
        dbuf[pl.ds(s * L, L)] = neg

    def s2(q2, wp):
        cid = cidbuf[pl.ds(q2 * L, L)]
        valid = (q2 * L + iota) < nacc_v
        ga = cid * P17
        jb = cid * L
        run = wp
        for h in range(2):
            gs, ms, bases = [], [], []
            for e in range(8):
                g = plsc.load_gather(rowpad, [ga + (h * 8 + e)])
                ms.append((g >= t0) & valid)
                gs.append(g)
            for e in range(8):
                bases.append(run)
                run = run + plsc.all_reduce_population_count(ms[e])
            for e in range(8):
                pos = jnp.minimum(
                    bases[e] + plsc.cumsum(ms[e].astype(jnp.int32)) - 1,
                    CAP - 1)
                plsc.store_scatter(dbuf, [pos], gs[e], mask=ms[e])
                plsc.store_scatter(jbuf, [pos], jb + (h * 8 + e),
                                   mask=ms[e])
        return run

    wp = lax.fori_loop(0, ng, s2, jnp.zeros((L,), jnp.int32))
    wps = jnp.max(wp)
    ns = (wps + (L - 1)) // L

    def _extract_sort():
        s0 = jnp.sort(dbuf[pl.ds(0, L)])
        s1_ = jnp.sort(dbuf[pl.ds(L, L)])
        s2_ = jnp.sort(dbuf[pl.ds(2 * L, L)])
        s3 = jnp.sort(dbuf[pl.ds(3 * L, L)])
        l0, h0 = _merge2_asc(s0, s1_)
        l1, h1 = _merge2_asc(s2_, s3)
        M0u = jnp.maximum(l0, h1[::-1])
        M1u = jnp.maximum(h0, l1[::-1])
        au = _bitonic_clean_asc(jnp.minimum(M0u, M1u))
        t20v = _shuf(au, jnp.full((L,), 12, jnp.int32))
        cnt = jnp.zeros((L,), jnp.int32)
        for s in range(4):
            cnt = cnt + plsc.all_reduce_population_count(
                dbuf[pl.ds(s * L, L)] > t20v)
        return jnp.max(cnt), jnp.max(t20v)

    def _extract_iter():
        for s in range(NSLOT):
            wbuf[pl.ds(s * L, L)] = dbuf[pl.ds(s * L, L)]

        def xcond(st):
            removed, _, _ = st
            return removed < K

        def xbody(st):
            removed, _, _ = st

            def slotmax(s, acc):
                return jnp.maximum(acc, wbuf[pl.ds(s * L, L)])

            mv = lax.fori_loop(0, ns, slotmax, neg)
            sv = _lane_max(mv)

            def rem(s, cnt):
                d = wbuf[pl.ds(s * L, L)]
                m = d == sv
                wbuf[pl.ds(s * L, L)] = jnp.where(m, neg, d)
                return cnt + plsc.all_reduce_population_count(m)

            cntv = lax.fori_loop(0, ns, rem, jnp.zeros((L,), jnp.int32))
            return removed + jnp.max(cntv), removed, jnp.max(sv)

        removed, a0, t20s0 = lax.while_loop(
            xcond, xbody, (jnp.int32(0), jnp.int32(0), jnp.float32(NEGINF)))
        del removed
        return a0, t20s0

    a, t20s = lax.cond(wps <= 4 * L, _extract_sort, _extract_iter)
    need = K - a
    t20 = jnp.full((L,), t20s, jnp.float32)

    def tcond(st):
        taken, _ = st
        return taken < need

    def tbody(st):
        taken, jprev = st

        def slotmin(s, acc):
            d = dbuf[pl.ds(s * L, L)]
            j = jbuf[pl.ds(s * L, L)]
            m = (d == t20) & (j > jprev)
            return jnp.minimum(acc, jnp.where(m, j, MAXI))

        jm = lax.fori_loop(0, ns, slotmin, jnp.full((L,), MAXI, jnp.int32))
        return taken + 1, jnp.min(jm)

    _, jcut = lax.while_loop(tcond, tbody, (jnp.int32(0), jnp.int32(-1)))
    jcut_v = jnp.full((L,), jcut, jnp.int32)


    def outs(s, run):
        d = dbuf[pl.ds(s * L, L)]
        j = jbuf[pl.ds(s * L, L)]
        sel = (d > t20) | ((d == t20) & (j <= jcut_v))
        rank = run + plsc.cumsum(sel.astype(jnp.int32)) - 1
        pos = rank * PPW + r
        plsc.store_scatter(oidx, [pos], j, mask=sel)
        return run + plsc.all_reduce_population_count(sel)

    lax.fori_loop(0, ns, outs, jnp.zeros((L,), jnp.int32))


def _sc_topk_body(pd_hbm, idx_hbm, row0, row1, rowpad, dbuf, jbuf, wbuf,
                  cidbuf, oidx, sem0, sem1):
    wid = lax.axis_index("s") * NC + lax.axis_index("c")
    b = wid // 2
    half = wid % 2
    base = half * PPW
    npair = PPW // 2

    for s in range(NSLOT):
        cidbuf[pl.ds(s * L, L)] = jnp.zeros((L,), jnp.int32)

    pltpu.async_copy(pd_hbm.at[b, base + 0], row0, sem0)
    pltpu.async_copy(pd_hbm.at[b, base + 1], row1, sem1)

    def pair(p, _):
        r0 = 2 * p
        r1 = 2 * p + 1
        pltpu.make_async_copy(pd_hbm.at[b, base + r0], row0, sem0).wait()
        _process_row(row0, rowpad, cidbuf, r0, dbuf, jbuf, wbuf, oidx)

        @pl.when(p < npair - 1)
        def _():
            pltpu.async_copy(pd_hbm.at[b, base + r0 + 2], row0, sem0)

        pltpu.make_async_copy(pd_hbm.at[b, base + r1], row1, sem1).wait()
        _process_row(row1, rowpad, cidbuf, r1, dbuf, jbuf, wbuf, oidx)

        @pl.when(p < npair - 1)
        def _():
            pltpu.async_copy(pd_hbm.at[b, base + r1 + 2], row1, sem1)

        return 0

    lax.fori_loop(0, npair, pair, 0)
    pltpu.sync_copy(oidx, idx_hbm.at[b, half])


def _sc_topk(pd):
    mesh = plsc.VectorSubcoreMesh(core_axis_name="c", subcore_axis_name="s")
    f = pl.kernel(
        _sc_topk_body,
        out_type=jax.ShapeDtypeStruct((B, 2, K * PPW), jnp.int32),
        mesh=mesh,
        compiler_params=pltpu.CompilerParams(needs_layout_passes=False),
        scratch_types=[
            pltpu.VMEM((N,), jnp.float32),
            pltpu.VMEM((N,), jnp.float32),
            pltpu.VMEM((N // L * P17,), jnp.float32),
            pltpu.VMEM((CAP,), jnp.float32),
            pltpu.VMEM((CAP,), jnp.int32),
            pltpu.VMEM((CAP,), jnp.float32),
            pltpu.VMEM((CAP,), jnp.int32),
            pltpu.VMEM((PPW * K,), jnp.int32),
            pltpu.SemaphoreType.DMA,
            pltpu.SemaphoreType.DMA,
        ],
    )
    return f(pd)


def _sc_gather_reduce(v_hbm, ut_hbm, idx_hbm,
                      mx_hbm, mn_hbm, part_hbm, cnt_hbm,
                      vblk, idxv, ucbuf, omx, omn, acc, cntb):
    wid = lax.axis_index("s") * NC + lax.axis_index("c")
    b = wid // 2
    half = wid % 2
    base = half * PPW

    def _z(i, _):
        acc[pl.ds(i * L, L)] = jnp.zeros((L,), jnp.float32)
        return 0
    lax.fori_loop(0, EMBED, _z, 0)

    def _zc(i, _):
        cntb[pl.ds(i * L, L)] = jnp.zeros((L,), jnp.float32)
        return 0
    lax.fori_loop(0, N // L, _zc, 0)

    pltpu.sync_copy(idx_hbm.at[b, half], idxv)

    ones = jnp.ones((L,), jnp.float32)

    for cb in range(NCB):
        pltpu.sync_copy(v_hbm.at[b, cb], vblk)
        for ch in range(NCH):
            noff = ch * NCHUNK
            pltpu.sync_copy(
                ut_hbm.at[b, cb, :, pl.ds(base + noff, NCHUNK)], ucbuf)

            def grp_body(gi, _, cb=cb, noff=noff):
                jv = [idxv[pl.ds(k * PPW + noff + gi * L, L)]
                      for k in range(K)]
                if cb == 0:
                    for k in range(K):
                        plsc.addupdate_scatter(cntb, [jv[k]], ones)

                @plsc.parallel_loop(0, CB, unroll=2)
                def c_body(c):
                    cbase = jnp.full((L,), c * N, jnp.int32)
                    u = ucbuf[c, pl.ds(gi * L, L)]
                    cr = jnp.zeros((L,), jnp.float32)
                    mx = jnp.full((L,), -jnp.inf, jnp.float32)
                    mn = jnp.full((L,), jnp.inf, jnp.float32)
                    for k in range(K):
                        g = plsc.load_gather(vblk, [jv[k] + cbase])
                        cr = cr + u * g
                        mx = jnp.maximum(mx, g)
                        mn = jnp.minimum(mn, g)
                    arow = (cb * CB + c) * L
                    acc[pl.ds(arow, L)] = acc[pl.ds(arow, L)] + cr
                    omx[c, pl.ds(gi * L, L)] = mx
                    omn[c, pl.ds(gi * L, L)] = mn

                return 0

            lax.fori_loop(0, NGRP, grp_body, 0)
            pltpu.sync_copy(
                omx, mx_hbm.at[b, cb, :, pl.ds(base + noff, NCHUNK)])
            pltpu.sync_copy(
                omn, mn_hbm.at[b, cb, :, pl.ds(base + noff, NCHUNK)])

    pltpu.sync_copy(acc, part_hbm.at[wid])
    pltpu.sync_copy(cntb, cnt_hbm.at[wid])


def _gather_reduce(v_blk, ut_blk, idx):
    mesh = plsc.VectorSubcoreMesh(core_axis_name="c", subcore_axis_name="s")
    f = pl.kernel(
        _sc_gather_reduce,
        out_type=[
            jax.ShapeDtypeStruct((B, NCB, CB, N), jnp.float32),
            jax.ShapeDtypeStruct((B, NCB, CB, N), jnp.float32),
            jax.ShapeDtypeStruct((NW, EMBED * L), jnp.float32),
            jax.ShapeDtypeStruct((NW, N), jnp.float32),
        ],
        mesh=mesh,
        compiler_params=pltpu.CompilerParams(needs_layout_passes=False),
        scratch_types=[
            pltpu.VMEM((N * CB,), jnp.float32),
            pltpu.VMEM((PPW * K,), jnp.int32),
            pltpu.VMEM((CB, NCHUNK), jnp.float32),
            pltpu.VMEM((CB, NCHUNK), jnp.float32),
            pltpu.VMEM((CB, NCHUNK), jnp.float32),
            pltpu.VMEM((EMBED * L,), jnp.float32),
            pltpu.VMEM((N,), jnp.float32),
        ],
    )
    return f(v_blk, ut_blk, idx)


def _combine_body(u_ref, vmax_ref, vmin_ref, g_ref, c_ref, o_ref):
    u = u_ref[...]
    g = g_ref[...]
    c = c_ref[...]
    hmax = (u + vmax_ref[...]) * g + c
    hmin = (u + vmin_ref[...]) * g + c

    def hswish(y):
        return y * jnp.clip(y + 3.0, 0.0, 6.0) * (1.0 / 6.0)

    o_ref[...] = jnp.maximum(hswish(hmax), hswish(hmin))


def _combine(UT, VmaxT, VminT, g, c):
    Bb, E, Nn = UT.shape
    TN = 512
    grid = (Bb, Nn // TN)
    in_spec = pl.BlockSpec((1, E, TN), lambda b, i: (b, 0, i))
    vec_spec = pl.BlockSpec((E, 1), lambda b, i: (0, 0))
    out_spec = pl.BlockSpec((1, E, TN), lambda b, i: (b, 0, i))
    f = pl.pallas_call(
        lambda u, vx, vn, gg, cc, o: _combine_body(
            u.at[0], vx.at[0], vn.at[0], gg, cc, o.at[0]),
        grid=grid,
        in_specs=[in_spec, in_spec, in_spec, vec_spec, vec_spec],
        out_specs=out_spec,
        out_shape=jax.ShapeDtypeStruct((Bb, E, Nn), jnp.float32),
    )
    return f(UT, VmaxT, VminT, g.reshape(E, 1), c.reshape(E, 1))


def kernel(xyz, W, gamma, beta):
    ut, vt, xx, su, su2 = _prep(xyz, W)
    pd = _pairwise(xyz, xx)
    idx = _sc_topk(pd)

    v_blk = vt.reshape(B, NCB, CB * N)
    ut_blk = ut.reshape(B, NCB, CB, N)
    mx, mn, part, cntw = _gather_reduce(v_blk, ut_blk, idx)

    cntb = (cntw[0::2, :] + cntw[1::2, :]).reshape(B, 1, N)
    cv, cv2 = _cstats(cntb, vt)
    s_S1 = jnp.sum(cv[:, :, 0], axis=0)
    s_S2 = jnp.sum(cv2[:, :, 0], axis=0)
    s_US1 = jnp.sum(part.reshape(NW, EMBED, L), axis=(0, 2))
    s_U = jnp.sum(su[:, 0], axis=0)
    s_U2 = jnp.sum(su2[:, 0], axis=0)
    cnt = B * N * K
    mean = (K * s_U + s_S1) / cnt
    var = (K * s_U2 + 2.0 * s_US1 + s_S2) / cnt - mean * mean
    g = gamma / jnp.sqrt(var + EPS)
    c = beta - g * mean

    x = _combine(ut, mx.reshape(B, EMBED, N), mn.reshape(B, EMBED, N), g, c)
    return (xyz, x)


# speedup vs baseline: 2.2754x; 1.2741x over previous
import functools

import jax
import jax.numpy as jnp
from jax import lax
from jax.experimental import pallas as pl
from jax.experimental.pallas import tpu as pltpu
from jax.experimental.pallas import tpu_sc as plsc

K = 20
EMBED = 128
EPS = 1e-5

NC, NS, L = 2, 16, 16
NW = NC * NS
B, N = 16, 2048
CB = 32
NCB = EMBED // CB
PPW = (B * N) // NW
NCHUNK = 256
NGRP = NCHUNK // L
NCH = PPW // NCHUNK
NSTAT = 5
CAP = 128
NSLOT = CAP // L
NEGINF = float("-inf")
MAXI = 2**30


def _prep_body(xyz_ref, w_ref, ut_ref, v_ref, xx_ref, su_ref, su2_ref):
    x = xyz_ref[0]
    w = w_ref[...]
    w1 = w[:, 0:3]
    w2 = w[:, 3:6]
    dn = (((1,), (1,)), ((), ()))
    ut = lax.dot_general(w1 - w2, x, dn, preferred_element_type=jnp.float32)
    ut_ref[0] = ut
    v_ref[0] = lax.dot_general(w2, x, dn, preferred_element_type=jnp.float32)
    xx_ref[0] = jnp.sum(x * x, axis=1).reshape(1, N)
    su_ref[0] = jnp.sum(ut, axis=1).reshape(1, EMBED)
    su2_ref[0] = jnp.sum(ut * ut, axis=1).reshape(1, EMBED)


def _prep(xyz, W):
    f = pl.pallas_call(
        _prep_body,
        grid=(B,),
        in_specs=[
            pl.BlockSpec((1, N, 3), lambda b: (b, 0, 0)),
            pl.BlockSpec((EMBED, 6), lambda b: (0, 0)),
        ],
        out_specs=[
            pl.BlockSpec((1, EMBED, N), lambda b: (b, 0, 0)),
            pl.BlockSpec((1, EMBED, N), lambda b: (b, 0, 0)),
            pl.BlockSpec((1, 1, N), lambda b: (b, 0, 0)),
            pl.BlockSpec((1, 1, EMBED), lambda b: (b, 0, 0)),
            pl.BlockSpec((1, 1, EMBED), lambda b: (b, 0, 0)),
        ],
        out_shape=[
            jax.ShapeDtypeStruct((B, EMBED, N), jnp.float32),
            jax.ShapeDtypeStruct((B, EMBED, N), jnp.float32),
            jax.ShapeDtypeStruct((B, 1, N), jnp.float32),
            jax.ShapeDtypeStruct((B, 1, EMBED), jnp.float32),
            jax.ShapeDtypeStruct((B, 1, EMBED), jnp.float32),
        ],
    )
    return f(xyz, W)


def _cstats_body(cnt_ref, v_ref, cv_ref, cv2_ref):
    cnt = cnt_ref[0]
    vt = v_ref[0]
    dn = (((1,), (1,)), ((), ()))
    cv_ref[0] = lax.dot_general(vt, cnt, dn,
                                preferred_element_type=jnp.float32)
    cv2_ref[0] = lax.dot_general(vt * vt, cnt, dn,
                                 preferred_element_type=jnp.float32)


def _cstats(cnt, vt):
    f = pl.pallas_call(
        _cstats_body,
        grid=(B,),
        in_specs=[
            pl.BlockSpec((1, 1, N), lambda b: (b, 0, 0)),
            pl.BlockSpec((1, EMBED, N), lambda b: (b, 0, 0)),
        ],
        out_specs=[
            pl.BlockSpec((1, EMBED, 1), lambda b: (b, 0, 0)),
            pl.BlockSpec((1, EMBED, 1), lambda b: (b, 0, 0)),
        ],
        out_shape=[
            jax.ShapeDtypeStruct((B, EMBED, 1), jnp.float32),
            jax.ShapeDtypeStruct((B, EMBED, 1), jnp.float32),
        ],
    )
    return f(cnt, vt)


TR = 256


def _pd_body(xt_ref, xf_ref, xxc_ref, xxr_ref, o_ref):
    xt = xt_ref[0]
    xf = xf_ref[0]
    dn = (((1,), (1,)), ((), ()))
    mm = lax.dot_general(xt, xf, dn, preferred_element_type=jnp.float32)
    inner = -2.0 * mm
    o_ref[0] = (-xxc_ref[0]) - inner - xxr_ref[0]


def _pairwise(xyz, xx):
    xxc = xx.reshape(B, N, 1)
    f = pl.pallas_call(
        _pd_body,
        grid=(B, N // TR),
        in_specs=[
            pl.BlockSpec((1, TR, 3), lambda b, i: (b, i, 0)),
            pl.BlockSpec((1, N, 3), lambda b, i: (b, 0, 0)),
            pl.BlockSpec((1, TR, 1), lambda b, i: (b, i, 0)),
            pl.BlockSpec((1, 1, N), lambda b, i: (b, 0, 0)),
        ],
        out_specs=pl.BlockSpec((1, TR, N), lambda b, i: (b, i, 0)),
        out_shape=jax.ShapeDtypeStruct((B, N, N), jnp.float32),
    )
    return f(xyz, xyz, xxc, xx)


def _shuf(v, idx):
    return jnp.take_along_axis(v, idx, axis=0, mode="promise_in_bounds")


def _lane_max(v):
    iota = lax.iota(jnp.int32, L)
    for dist in (8, 4, 2, 1):
        v = jnp.maximum(v, _shuf(v, jnp.bitwise_xor(iota, dist)))
    return v


def _lane_min_i(v):
    iota = lax.iota(jnp.int32, L)
    for dist in (8, 4, 2, 1):
        v = jnp.minimum(v, _shuf(v, jnp.bitwise_xor(iota, dist)))
    return v


def _bitonic_clean_asc(v):
    iota = lax.iota(jnp.int32, L)
    for dist in (8, 4, 2, 1):
        w = _shuf(v, jnp.bitwise_xor(iota, dist))
        up = (jnp.bitwise_and(iota, dist) == 0)
        v = jnp.where(up, jnp.minimum(v, w), jnp.maximum(v, w))
    return v


def _merge2_asc(a, b):
    bl = b[::-1]
    lo = _bitonic_clean_asc(jnp.minimum(a, bl))
    hi = _bitonic_clean_asc(jnp.maximum(a, bl))
    return lo, hi


P17 = L + 1


def _process_row(rowref, cidbuf, r, dbuf, jbuf, wbuf, oidx):
    iota = lax.iota(jnp.int32, L)
    neg = jnp.full((L,), NEGINF, jnp.float32)
    iota17 = iota * P17
    nlim = jnp.full((L,), N - 1, jnp.int32)

    cms = []
    for q in range(8):
        base = q * (L * P17) + iota17
        cm = neg
        for e in range(P17):
            a = jnp.minimum(base + e, nlim)
            cm = jnp.maximum(cm, plsc.load_gather(rowref, [a]))
        cms.append(cm)
    cms[7] = jnp.where(iota < 9, cms[7], neg)

    M0 = jnp.maximum(jnp.maximum(cms[0], cms[1]), jnp.maximum(cms[2], cms[3]))
    M1 = jnp.maximum(jnp.maximum(cms[4], cms[5]), jnp.maximum(cms[6], cms[7]))
    sA = jnp.sort(M0)
    sB = jnp.sort(M1)
    lo = jnp.minimum(sA, sB[::-1])
    lo = _bitonic_clean_asc(lo)
    t0 = _shuf(lo, jnp.full((L,), 11, jnp.int32))

    cw = jnp.zeros((L,), jnp.int32)
    for q in range(8):
        m = cms[q] >= t0
        pos = jnp.minimum(cw + plsc.cumsum(m.astype(jnp.int32)) - 1, CAP - 1)
        plsc.store_scatter(cidbuf, [pos], q * L + iota, mask=m)
        cw = cw + plsc.all_reduce_population_count(m)
    nacc_v = cw
    ng = (jnp.max(cw) + (L - 1)) // L

    for s in range(NSLOT):
        dbuf[pl.ds(s * L, L)] = neg

    def s2(q2, wp):
        cid = cidbuf[pl.ds(q2 * L, L)]
        valid = (q2 * L + iota) < nacc_v
        ga = cid * P17
        run = wp
        for grp in (range(0, 9), range(9, P17)):
            gs, ms, adr = [], [], []
            for e in grp:
                a = ga + e
                inb = a < N
                a = jnp.minimum(a, nlim)
                g = plsc.load_gather(rowref, [a])
                ms.append((g >= t0) & valid & inb)
                gs.append(g)
                adr.append(a)
            for i in range(len(gs)):
                bases_i = run
                run = run + plsc.all_reduce_population_count(ms[i])
                pos = jnp.minimum(
                    bases_i + plsc.cumsum(ms[i].astype(jnp.int32)) - 1,
                    CAP - 1)
                plsc.store_scatter(dbuf, [pos], gs[i], mask=ms[i])
                plsc.store_scatter(jbuf, [pos], adr[i], mask=ms[i])
        return run

    wp = lax.fori_loop(0, ng, s2, jnp.zeros((L,), jnp.int32))
    wps = jnp.max(wp)
    ns = (wps + (L - 1)) // L

    def _extract_sort():
        s0 = jnp.sort(dbuf[pl.ds(0, L)])
        s1_ = jnp.sort(dbuf[pl.ds(L, L)])
        s2_ = jnp.sort(dbuf[pl.ds(2 * L, L)])
        s3 = jnp.sort(dbuf[pl.ds(3 * L, L)])
        l0, h0 = _merge2_asc(s0, s1_)
        l1, h1 = _merge2_asc(s2_, s3)
        M0u = jnp.maximum(l0, h1[::-1])
        M1u = jnp.maximum(h0, l1[::-1])
        au = _bitonic_clean_asc(jnp.minimum(M0u, M1u))
        t20v = _shuf(au, jnp.full((L,), 12, jnp.int32))
        cnt = jnp.zeros((L,), jnp.int32)
        for s in range(4):
            cnt = cnt + plsc.all_reduce_population_count(
                dbuf[pl.ds(s * L, L)] > t20v)
        return jnp.max(cnt), jnp.max(t20v)

    def _extract_iter():
        for s in range(NSLOT):
            wbuf[pl.ds(s * L, L)] = dbuf[pl.ds(s * L, L)]

        def xcond(st):
            removed, _, _ = st
            return removed < K

        def xbody(st):
            removed, _, _ = st

            def slotmax(s, acc):
                return jnp.maximum(acc, wbuf[pl.ds(s * L, L)])

            mv = lax.fori_loop(0, ns, slotmax, neg)
            sv = _lane_max(mv)

            def rem(s, cnt):
                d = wbuf[pl.ds(s * L, L)]
                m = d == sv
                wbuf[pl.ds(s * L, L)] = jnp.where(m, neg, d)
                return cnt + plsc.all_reduce_population_count(m)

            cntv = lax.fori_loop(0, ns, rem, jnp.zeros((L,), jnp.int32))
            return removed + jnp.max(cntv), removed, jnp.max(sv)

        removed, a0, t20s0 = lax.while_loop(
            xcond, xbody, (jnp.int32(0), jnp.int32(0), jnp.float32(NEGINF)))
        del removed
        return a0, t20s0

    a, t20s = lax.cond(wps <= 4 * L, _extract_sort, _extract_iter)
    need = K - a
    t20 = jnp.full((L,), t20s, jnp.float32)

    def tcond(st):
        taken, _ = st
        return taken < need

    def tbody(st):
        taken, jprev = st

        def slotmin(s, acc):
            d = dbuf[pl.ds(s * L, L)]
            j = jbuf[pl.ds(s * L, L)]
            m = (d == t20) & (j > jprev)
            return jnp.minimum(acc, jnp.where(m, j, MAXI))

        jm = lax.fori_loop(0, ns, slotmin, jnp.full((L,), MAXI, jnp.int32))
        return taken + 1, jnp.min(jm)

    _, jcut = lax.while_loop(tcond, tbody, (jnp.int32(0), jnp.int32(-1)))
    jcut_v = jnp.full((L,), jcut, jnp.int32)


    def outs(s, run):
        d = dbuf[pl.ds(s * L, L)]
        j = jbuf[pl.ds(s * L, L)]
        sel = (d > t20) | ((d == t20) & (j <= jcut_v))
        rank = run + plsc.cumsum(sel.astype(jnp.int32)) - 1
        pos = rank * PPW + r
        plsc.store_scatter(oidx, [pos], j, mask=sel)
        return run + plsc.all_reduce_population_count(sel)

    lax.fori_loop(0, ns, outs, jnp.zeros((L,), jnp.int32))


def _sc_topk_body(pd_hbm, idx_hbm, row0, row1, dbuf, jbuf, wbuf,
                  cidbuf, oidx, sem0, sem1):
    wid = lax.axis_index("s") * NC + lax.axis_index("c")
    b = wid // 2
    half = wid % 2
    base = half * PPW
    npair = PPW // 2

    for s in range(NSLOT):
        cidbuf[pl.ds(s * L, L)] = jnp.zeros((L,), jnp.int32)

    pltpu.async_copy(pd_hbm.at[b, base + 0], row0, sem0)
    pltpu.async_copy(pd_hbm.at[b, base + 1], row1, sem1)

    def pair(p, _):
        r0 = 2 * p
        r1 = 2 * p + 1
        pltpu.make_async_copy(pd_hbm.at[b, base + r0], row0, sem0).wait()
        _process_row(row0, cidbuf, r0, dbuf, jbuf, wbuf, oidx)

        @pl.when(p < npair - 1)
        def _():
            pltpu.async_copy(pd_hbm.at[b, base + r0 + 2], row0, sem0)

        pltpu.make_async_copy(pd_hbm.at[b, base + r1], row1, sem1).wait()
        _process_row(row1, cidbuf, r1, dbuf, jbuf, wbuf, oidx)

        @pl.when(p < npair - 1)
        def _():
            pltpu.async_copy(pd_hbm.at[b, base + r1 + 2], row1, sem1)

        return 0

    lax.fori_loop(0, npair, pair, 0)
    pltpu.sync_copy(oidx, idx_hbm.at[b, half])


def _sc_topk(pd):
    mesh = plsc.VectorSubcoreMesh(core_axis_name="c", subcore_axis_name="s")
    f = pl.kernel(
        _sc_topk_body,
        out_type=jax.ShapeDtypeStruct((B, 2, K * PPW), jnp.int32),
        mesh=mesh,
        compiler_params=pltpu.CompilerParams(needs_layout_passes=False),
        scratch_types=[
            pltpu.VMEM((N,), jnp.float32),
            pltpu.VMEM((N,), jnp.float32),
            pltpu.VMEM((CAP,), jnp.float32),
            pltpu.VMEM((CAP,), jnp.int32),
            pltpu.VMEM((CAP,), jnp.float32),
            pltpu.VMEM((CAP,), jnp.int32),
            pltpu.VMEM((PPW * K,), jnp.int32),
            pltpu.SemaphoreType.DMA,
            pltpu.SemaphoreType.DMA,
        ],
    )
    return f(pd)


def _sc_gather_reduce(v_hbm, ut_hbm, idx_hbm,
                      mx_hbm, mn_hbm, part_hbm, cnt_hbm,
                      vblk, idxv, ucbuf, omx, omn, acc, cntb):
    wid = lax.axis_index("s") * NC + lax.axis_index("c")
    b = wid // 2
    half = wid % 2
    base = half * PPW

    def _z(i, _):
        acc[pl.ds(i * L, L)] = jnp.zeros((L,), jnp.float32)
        return 0
    lax.fori_loop(0, EMBED, _z, 0)

    def _zc(i, _):
        cntb[pl.ds(i * L, L)] = jnp.zeros((L,), jnp.float32)
        return 0
    lax.fori_loop(0, N // L, _zc, 0)

    pltpu.sync_copy(idx_hbm.at[b, half], idxv)

    ones = jnp.ones((L,), jnp.float32)

    for cb in range(NCB):
        pltpu.sync_copy(v_hbm.at[b, cb], vblk)
        for ch in range(NCH):
            noff = ch * NCHUNK
            pltpu.sync_copy(
                ut_hbm.at[b, cb, :, pl.ds(base + noff, NCHUNK)], ucbuf)

            def grp_body(gi, _, cb=cb, noff=noff):
                jv = [idxv[pl.ds(k * PPW + noff + gi * L, L)]
                      for k in range(K)]
                if cb == 0:
                    for k in range(K):
                        plsc.addupdate_scatter(cntb, [jv[k]], ones)

                @plsc.parallel_loop(0, CB, unroll=2)
                def c_body(c):
                    cbase = jnp.full((L,), c * N, jnp.int32)
                    u = ucbuf[c, pl.ds(gi * L, L)]
                    cr = jnp.zeros((L,), jnp.float32)
                    mx = jnp.full((L,), -jnp.inf, jnp.float32)
                    mn = jnp.full((L,), jnp.inf, jnp.float32)
                    for k in range(K):
                        g = plsc.load_gather(vblk, [jv[k] + cbase])
                        cr = cr + u * g
                        mx = jnp.maximum(mx, g)
                        mn = jnp.minimum(mn, g)
                    arow = (cb * CB + c) * L
                    acc[pl.ds(arow, L)] = acc[pl.ds(arow, L)] + cr
                    omx[c, pl.ds(gi * L, L)] = mx
                    omn[c, pl.ds(gi * L, L)] = mn

                return 0

            lax.fori_loop(0, NGRP, grp_body, 0)
            pltpu.sync_copy(
                omx, mx_hbm.at[b, cb, :, pl.ds(base + noff, NCHUNK)])
            pltpu.sync_copy(
                omn, mn_hbm.at[b, cb, :, pl.ds(base + noff, NCHUNK)])

    pltpu.sync_copy(acc, part_hbm.at[wid])
    pltpu.sync_copy(cntb, cnt_hbm.at[wid])


def _gather_reduce(v_blk, ut_blk, idx):
    mesh = plsc.VectorSubcoreMesh(core_axis_name="c", subcore_axis_name="s")
    f = pl.kernel(
        _sc_gather_reduce,
        out_type=[
            jax.ShapeDtypeStruct((B, NCB, CB, N), jnp.float32),
            jax.ShapeDtypeStruct((B, NCB, CB, N), jnp.float32),
            jax.ShapeDtypeStruct((NW, EMBED * L), jnp.float32),
            jax.ShapeDtypeStruct((NW, N), jnp.float32),
        ],
        mesh=mesh,
        compiler_params=pltpu.CompilerParams(needs_layout_passes=False),
        scratch_types=[
            pltpu.VMEM((N * CB,), jnp.float32),
            pltpu.VMEM((PPW * K,), jnp.int32),
            pltpu.VMEM((CB, NCHUNK), jnp.float32),
            pltpu.VMEM((CB, NCHUNK), jnp.float32),
            pltpu.VMEM((CB, NCHUNK), jnp.float32),
            pltpu.VMEM((EMBED * L,), jnp.float32),
            pltpu.VMEM((N,), jnp.float32),
        ],
    )
    return f(v_blk, ut_blk, idx)


def _combine_body(u_ref, vmax_ref, vmin_ref, g_ref, c_ref, o_ref):
    u = u_ref[...]
    g = g_ref[...]
    c = c_ref[...]
    hmax = (u + vmax_ref[...]) * g + c
    hmin = (u + vmin_ref[...]) * g + c

    def hswish(y):
        return y * jnp.clip(y + 3.0, 0.0, 6.0) * (1.0 / 6.0)

    o_ref[...] = jnp.maximum(hswish(hmax), hswish(hmin))


def _combine(UT, VmaxT, VminT, g, c):
    Bb, E, Nn = UT.shape
    TN = 512
    grid = (Bb, Nn // TN)
    in_spec = pl.BlockSpec((1, E, TN), lambda b, i: (b, 0, i))
    vec_spec = pl.BlockSpec((E, 1), lambda b, i: (0, 0))
    out_spec = pl.BlockSpec((1, E, TN), lambda b, i: (b, 0, i))
    f = pl.pallas_call(
        lambda u, vx, vn, gg, cc, o: _combine_body(
            u.at[0], vx.at[0], vn.at[0], gg, cc, o.at[0]),
        grid=grid,
        in_specs=[in_spec, in_spec, in_spec, vec_spec, vec_spec],
        out_specs=out_spec,
        out_shape=jax.ShapeDtypeStruct((Bb, E, Nn), jnp.float32),
    )
    return f(UT, VmaxT, VminT, g.reshape(E, 1), c.reshape(E, 1))


def kernel(xyz, W, gamma, beta):
    ut, vt, xx, su, su2 = _prep(xyz, W)
    pd = _pairwise(xyz, xx)
    idx = _sc_topk(pd)

    v_blk = vt.reshape(B, NCB, CB * N)
    ut_blk = ut.reshape(B, NCB, CB, N)
    mx, mn, part, cntw = _gather_reduce(v_blk, ut_blk, idx)

    cntb = (cntw[0::2, :] + cntw[1::2, :]).reshape(B, 1, N)
    cv, cv2 = _cstats(cntb, vt)
    s_S1 = jnp.sum(cv[:, :, 0], axis=0)
    s_S2 = jnp.sum(cv2[:, :, 0], axis=0)
    s_US1 = jnp.sum(part.reshape(NW, EMBED, L), axis=(0, 2))
    s_U = jnp.sum(su[:, 0], axis=0)
    s_U2 = jnp.sum(su2[:, 0], axis=0)
    cnt = B * N * K
    mean = (K * s_U + s_S1) / cnt
    var = (K * s_U2 + 2.0 * s_US1 + s_S2) / cnt - mean * mean
    g = gamma / jnp.sqrt(var + EPS)
    c = beta - g * mean

    x = _combine(ut, mx.reshape(B, EMBED, N), mn.reshape(B, EMBED, N), g, c)
    return (xyz, x)
